# baseline (speedup 1.0000x reference)
import functools

import numpy as np
import jax
import jax.numpy as jnp
from jax import lax
from jax.experimental import pallas as pl
from jax.experimental.pallas import tpu as pltpu
from jax.experimental.pallas import tpu_sc as plsc

N = 10000
E = 160000
HID = 128
HEADS = 16
DH = HID // HEADS
SHAPE_DIM = 16
EDGE_DIM = 4
NG = 20
R_MIN, R_MAX = 0.0, 10.0
RSQRT_DH = float(1.0 / np.sqrt(DH))
STEP = (R_MAX - R_MIN) / (NG - 1)
COEFF = -0.5 / STEP**2

NODE_BLK = 1000
EDGE_BLK = 1280

_INTERPRET = False

TD_W = 384
TS_W = 256


def _ln_mxu(hdn, g, be):
    o1 = jnp.full((HID, HEADS), 1.0 / HID, jnp.float32)
    o2 = jnp.full((HEADS, HID), 1.0 / HEADS, jnp.float32)
    mu = (hdn @ o1) @ o2
    ex2 = ((hdn * hdn) @ o1) @ o2
    var = ex2 - mu * mu
    return (hdn - mu) * jax.lax.rsqrt(var + 1e-5) * g + be


def _sel(rows, cols, fn):
    r = jax.lax.broadcasted_iota(jnp.int32, (rows, cols), 0)
    c = jax.lax.broadcasted_iota(jnp.int32, (rows, cols), 1)
    return fn(r, c).astype(jnp.float32)


def _full_spec(a):
    nd = a.ndim
    return pl.BlockSpec(a.shape, lambda i, *, _nd=nd: (0,) * _nd)


def _head_sum_matrix():
    j = jax.lax.broadcasted_iota(jnp.int32, (HID, HEADS), 0)
    h = jax.lax.broadcasted_iota(jnp.int32, (HID, HEADS), 1)
    return (j // DH == h).astype(jnp.float32)


def _tables_math(h, inv, wk, wv, wq):
    A_k = h @ wk[0] + inv @ wk[1] + wk[2]
    A_v = h @ wv[0] + inv @ wv[1] + wv[2]
    B_k = h @ wk[3]
    B_v = h @ wv[3]
    hdnq = h @ wq[0] + wq[1]
    q = jnp.maximum(_ln_mxu(hdnq, wq[2], wq[3]), 0.0) @ wq[4] + wq[5]
    td = jnp.concatenate([A_k, A_v, q], axis=1)
    ts = jnp.concatenate([B_k, B_v], axis=1)
    return td, ts


def _node1_body(h_ref, inv_ref, *rest):
    (wk0, wk1, wk2, wk3, wv0, wv1, wv2, wv3,
     q0, q1, q2, q3, q4, q5, td_ref, ts_ref) = rest
    td, ts = _tables_math(
        h_ref[...], inv_ref[...],
        (wk0[...], wk1[...], wk2[...], wk3[...]),
        (wv0[...], wv1[...], wv2[...], wv3[...]),
        (q0[...], q1[...], q2[...], q3[...], q4[...], q5[...]))
    td_ref[...] = td
    ts_ref[...] = ts


def _node_tables(h, inv, wk, wv, wq):
    args = [h, inv, *wk, *wv, *wq]
    in_specs = [pl.BlockSpec((NODE_BLK, h.shape[1]), lambda i: (i, 0)),
                pl.BlockSpec((NODE_BLK, SHAPE_DIM), lambda i: (i, 0))]
    in_specs += [_full_spec(a) for a in args[2:]]
    return pl.pallas_call(
        _node1_body,
        grid=(N // NODE_BLK,),
        in_specs=in_specs,
        out_specs=[pl.BlockSpec((NODE_BLK, TD_W), lambda i: (i, 0)),
                   pl.BlockSpec((NODE_BLK, TS_W), lambda i: (i, 0))],
        out_shape=[jax.ShapeDtypeStruct((N, TD_W), jnp.float32),
                   jax.ShapeDtypeStruct((N, TS_W), jnp.float32)],
        interpret=_INTERPRET,
    )(*args)


def _edge_feats(xw, ea):
    rel = xw[:, 0:16] - xw[:, 16:32]
    d2 = (rel * rel) @ jnp.ones((16, NG), jnp.float32)
    dist = jnp.sqrt(d2 + 1e-12)
    offs = jax.lax.broadcasted_iota(jnp.int32, (1, NG), 1).astype(
        jnp.float32) * STEP
    df = jnp.exp(COEFF * (dist - offs) ** 2)
    ea_b = ea @ _sel(EDGE_DIM, NG * EDGE_DIM, lambda r, c: c // NG == r)
    df_b = df @ _sel(NG, NG * EDGE_DIM, lambda r, c: c % NG == r)
    return ea_b * df_b, rel


def _edge_mlp(ea, rf, gd, gs, off, w1a, w1r, g, be, w2, b2):
    hdn = ea @ w1a + rf @ w1r + gd[:, off:off + HID] + gs[:, off:off + HID]
    return jnp.maximum(_ln_mxu(hdn, g, be), 0.0) @ w2 + b2


def _edge1_body(gd_ref, gs_ref, xw_ref, ea_ref, ew_ref, *rest):
    (k_w1a, k_w1r, k_g, k_be, k_w2, k_b2,
     v_w1a, v_w1r, v_g, v_be, v_w2p, v_b2p, s1w_ref, s1d_ref) = rest
    gd = gd_ref[...]
    gs = gs_ref[...]
    ea = ea_ref[...]
    rf, _ = _edge_feats(xw_ref[...], ea)
    kk = _edge_mlp(ea, rf, gd, gs, 0, k_w1a[...], k_w1r[...], k_g[...],
                   k_be[...], k_w2[...], k_b2[...])
    q = gd[:, 256:384]
    logits = ((q * kk) @ _head_sum_matrix()) * RSQRT_DH
    ex = jnp.exp(logits)
    vt = _edge_mlp(ea, rf, gd, gs, HID, v_w1a[...], v_w1r[...], v_g[...],
                   v_be[...], v_w2p[...], v_b2p[...])
    ext = ex @ _sel(HEADS, HID, lambda r, c: c % HEADS == r)
    extw = ext * (ew_ref[...] @ jnp.ones((1, HID), jnp.float32))
    s1w_ref[...] = extw * vt
    s1d_ref[...] = ext


def _edge1(gd, gs, xw, ea, ew, wk, wv):
    args = [gd, gs, xw, ea, ew, *wk, *wv]
    in_specs = [pl.BlockSpec((EDGE_BLK, TD_W), lambda i: (i, 0)),
                pl.BlockSpec((EDGE_BLK, TS_W), lambda i: (i, 0)),
                pl.BlockSpec((EDGE_BLK, HID), lambda i: (i, 0)),
                pl.BlockSpec((EDGE_BLK, EDGE_DIM), lambda i: (i, 0)),
                pl.BlockSpec((EDGE_BLK, 1), lambda i: (i, 0))]
    in_specs += [_full_spec(a) for a in args[5:]]
    return pl.pallas_call(
        _edge1_body,
        grid=(E // EDGE_BLK,),
        in_specs=in_specs,
        out_specs=[pl.BlockSpec((EDGE_BLK, HID), lambda i: (i, 0)),
                   pl.BlockSpec((EDGE_BLK, HID), lambda i: (i, 0))],
        out_shape=[jax.ShapeDtypeStruct((E, HID), jnp.float32),
                   jax.ShapeDtypeStruct((E, HID), jnp.float32)],
        interpret=_INTERPRET,
    )(*args)


def _edge2_body(gd_ref, gs_ref, xw_ref, ea_ref, ew_ref, *rest):
    (k_w1a, k_w1r, k_g, k_be, k_w2, k_b2,
     v_w1a, v_w1r, v_g, v_be, v_w2, v_b2, s2_ref) = rest
    gd = gd_ref[...]
    gs = gs_ref[...]
    ea = ea_ref[...]
    rf, rel = _edge_feats(xw_ref[...], ea)
    kk = _edge_mlp(ea, rf, gd, gs, 0, k_w1a[...], k_w1r[...], k_g[...],
                   k_be[...], k_w2[...], k_b2[...])
    q = gd[:, 256:384]
    logits = ((q * kk) @ _head_sum_matrix()) * RSQRT_DH
    ex = jnp.exp(logits)
    v2 = _edge_mlp(ea, rf, gd, gs, HID, v_w1a[...], v_w1r[...], v_g[...],
                   v_be[...], v_w2[...], v_b2[...])
    vv = ex * (ew_ref[...] @ jnp.ones((1, HEADS), jnp.float32)) * v2
    vv3 = vv @ _sel(HEADS, 48, lambda r, c: c % HEADS == r)
    rel3 = rel @ _sel(16, 48, lambda r, c: c // HEADS == r)
    zpad = jnp.zeros((vv3.shape[0], 64), jnp.float32)
    s2_ref[...] = jnp.concatenate([vv3 * rel3, ex, zpad], axis=1)


def _edge2(gd, gs, xw, ea, ew, wk, wv):
    args = [gd, gs, xw, ea, ew, *wk, *wv]
    in_specs = [pl.BlockSpec((EDGE_BLK, TD_W), lambda i: (i, 0)),
                pl.BlockSpec((EDGE_BLK, TS_W), lambda i: (i, 0)),
                pl.BlockSpec((EDGE_BLK, HID), lambda i: (i, 0)),
                pl.BlockSpec((EDGE_BLK, EDGE_DIM), lambda i: (i, 0)),
                pl.BlockSpec((EDGE_BLK, 1), lambda i: (i, 0))]
    in_specs += [_full_spec(a) for a in args[5:]]
    return pl.pallas_call(
        _edge2_body,
        grid=(E // EDGE_BLK,),
        in_specs=in_specs,
        out_specs=pl.BlockSpec((EDGE_BLK, HID), lambda i: (i, 0)),
        out_shape=jax.ShapeDtypeStruct((E, HID), jnp.float32),
        interpret=_INTERPRET,
    )(*args)


def _node2_body(accw_ref, accd_ref, h_ref, inv_ref, *rest):
    (n_w1, n_b1, n_g, n_be, n_w2, n_b2,
     wk0, wk1, wk2, wk3, wv0, wv1, wv2, wv3,
     q0, q1, q2, q3, q4, q5, ho_ref, td_ref, ts_ref) = rest
    accw = accw_ref[...]
    h = h_ref[...]
    num = accw[0] + accw[1]
    accd = accd_ref[...]
    dent = accd[0] + accd[1]
    out_t = num / (dent + 1e-16)
    u = jnp.concatenate([out_t, h], axis=1)
    hdn = u @ n_w1[...] + n_b1[...]
    ho = jnp.maximum(_ln_mxu(hdn, n_g[...], n_be[...]), 0.0) @ n_w2[...] \
        + n_b2[...] + h
    ho_ref[...] = ho
    td, ts = _tables_math(
        ho, inv_ref[...],
        (wk0[...], wk1[...], wk2[...], wk3[...]),
        (wv0[...], wv1[...], wv2[...], wv3[...]),
        (q0[...], q1[...], q2[...], q3[...], q4[...], q5[...]))
    td_ref[...] = td
    ts_ref[...] = ts


def _node2(accw, accd, h, inv, wn, wk, wv, wq):
    args = [accw, accd, h, inv, *wn, *wk, *wv, *wq]
    in_specs = [pl.BlockSpec((2, NODE_BLK, HID), lambda i: (0, i, 0)),
                pl.BlockSpec((2, NODE_BLK, HID), lambda i: (0, i, 0)),
                pl.BlockSpec((NODE_BLK, HID), lambda i: (i, 0)),
                pl.BlockSpec((NODE_BLK, SHAPE_DIM), lambda i: (i, 0))]
    in_specs += [_full_spec(a) for a in args[4:]]
    return pl.pallas_call(
        _node2_body,
        grid=(N // NODE_BLK,),
        in_specs=in_specs,
        out_specs=[pl.BlockSpec((NODE_BLK, HID), lambda i: (i, 0)),
                   pl.BlockSpec((NODE_BLK, TD_W), lambda i: (i, 0)),
                   pl.BlockSpec((NODE_BLK, TS_W), lambda i: (i, 0))],
        out_shape=[jax.ShapeDtypeStruct((N, HID), jnp.float32),
                   jax.ShapeDtypeStruct((N, TD_W), jnp.float32),
                   jax.ShapeDtypeStruct((N, TS_W), jnp.float32)],
        interpret=_INTERPRET,
    )(*args)


def _tail_body(acc_ref, x_ref, se0_ref, se1_ref, se2_ref, wft_ref, wdt_ref,
               xo_ref):
    acc = acc_ref[...]
    x = x_ref[...]
    se = (se0_ref[...], se1_ref[...], se2_ref[...])
    den = acc[0, :, 48:64] + acc[1, :, 48:64]
    wft = wft_ref[...]
    wdt = wdt_ref[...]
    outs, Ps, Ds = [], [], []
    for c in range(3):
        num = acc[0, :, c * 16:(c + 1) * 16] + acc[1, :, c * 16:(c + 1) * 16]
        oc = num / (den + 1e-16)
        outs.append(oc)
        tmp = jnp.concatenate([x[:, c:c + 1], oc, se[c]], axis=1)
        Ps.append(tmp @ wft)
        Ds.append(tmp @ wdt)
    dot = Ps[0] * Ds[0] + Ps[1] * Ds[1] + Ps[2] * Ds[2]
    dsq = Ds[0] * Ds[0] + Ds[1] * Ds[1] + Ds[2] * Ds[2]
    coef = dot / (dsq + 1e-6)
    mask = dot >= 0.0
    deltas = []
    for c in range(3):
        neg = jnp.where(mask, Ps[c], Ps[c] - coef * Ds[c])
        res = 0.2 * Ps[c] + 0.8 * neg
        delta = jnp.mean(outs[c], axis=-1, keepdims=True) \
            + jnp.mean(res, axis=-1, keepdims=True)
        deltas.append(x[:, c:c + 1] + delta)
    xo_ref[...] = jnp.concatenate(deltas, axis=1)


def _tail(acc2, x, se0, se1, se2, wft, wdt):
    args = [acc2, x, se0, se1, se2, wft, wdt]
    in_specs = [pl.BlockSpec((2, NODE_BLK, HID), lambda i: (0, i, 0)),
                pl.BlockSpec((NODE_BLK, 3), lambda i: (i, 0)),
                pl.BlockSpec((NODE_BLK, 16), lambda i: (i, 0)),
                pl.BlockSpec((NODE_BLK, 16), lambda i: (i, 0)),
                pl.BlockSpec((NODE_BLK, 16), lambda i: (i, 0)),
                _full_spec(wft), _full_spec(wdt)]
    return pl.pallas_call(
        _tail_body,
        grid=(N // NODE_BLK,),
        in_specs=in_specs,
        out_specs=pl.BlockSpec((NODE_BLK, 3), lambda i: (i, 0)),
        out_shape=jax.ShapeDtypeStruct((N, 3), jnp.float32),
        interpret=_INTERPRET,
    )(*args)


CHUNK = 128
NROWS = E // CHUNK
NWORK = 32
ROWS_PER_W = -(-NROWS // NWORK)
NODES_PER_SUB = N // 16

_SC_MESH = plsc.VectorSubcoreMesh(core_axis_name="c", subcore_axis_name="s")


CHUNK2 = 64
NCHTOT = E // CHUNK2
NCH = -(-NCHTOT // NWORK) if NCHTOT % NWORK else NCHTOT // NWORK
NCH = 80


def _gather(td, ts, dst_m64, src_m64):
    @functools.partial(
        pl.kernel,
        out_type=[jax.ShapeDtypeStruct((E, TD_W), jnp.float32),
                  jax.ShapeDtypeStruct((E, TS_W), jnp.float32)],
        mesh=_SC_MESH,
        scratch_types=[pltpu.VMEM((NCH, CHUNK2), jnp.int32),
                       pltpu.VMEM((NCH, CHUNK2), jnp.int32),
                       pltpu.VMEM((2, CHUNK2, TD_W), jnp.float32),
                       pltpu.VMEM((2, CHUNK2, TS_W), jnp.float32),
                       pltpu.SemaphoreType.DMA,
                       pltpu.SemaphoreType.DMA,
                       pltpu.SemaphoreType.DMA,
                       pltpu.SemaphoreType.DMA],
    )
    def gk(td_hbm, ts_hbm, dm_hbm, sm_hbm, gd_hbm, gs_hbm,
           idx_d, idx_s, rows_d, rows_s, sd0, ss0, sd1, ss1):
        wid = lax.axis_index("s") * 2 + lax.axis_index("c")
        start = wid * NCH
        pltpu.sync_copy(dm_hbm.at[pl.ds(start, NCH)], idx_d)
        pltpu.sync_copy(sm_hbm.at[pl.ds(start, NCH)], idx_s)
        semd = (sd0, sd1)
        sems = (ss0, ss1)

        def issue(s, j):
            @pl.when((j < NCH) & (start + j < NCHTOT))
            def _():
                pltpu.async_copy(td_hbm.at[idx_d.at[j]], rows_d.at[s],
                                 semd[s])
                pltpu.async_copy(ts_hbm.at[idx_s.at[j]], rows_s.at[s],
                                 sems[s])

        def flush(s, j):
            @pl.when((j < NCH) & (start + j < NCHTOT))
            def _():
                pltpu.make_async_copy(td_hbm.at[idx_d.at[j]], rows_d.at[s],
                                      semd[s]).wait()
                pltpu.make_async_copy(ts_hbm.at[idx_s.at[j]], rows_s.at[s],
                                      sems[s]).wait()
                r = start + j
                pltpu.sync_copy(rows_d.at[s],
                                gd_hbm.at[pl.ds(r * CHUNK2, CHUNK2)])
                pltpu.sync_copy(rows_s.at[s],
                                gs_hbm.at[pl.ds(r * CHUNK2, CHUNK2)])

        issue(0, 0)

        @pl.loop(0, NCH // 2)
        def _(p):
            issue(1, 2 * p + 1)
            flush(0, 2 * p)
            issue(0, 2 * p + 2)
            flush(1, 2 * p + 1)

    return gk(td, ts, dst_m64, src_m64)


PCHUNK = CHUNK // 8


def _gather_rel(xpad, dst_m, src_m):
    @functools.partial(
        pl.kernel,
        out_type=jax.ShapeDtypeStruct((E, HID), jnp.float32),
        mesh=_SC_MESH,
        scratch_types=[pltpu.VMEM((ROWS_PER_W, CHUNK), jnp.int32),
                       pltpu.VMEM((ROWS_PER_W, CHUNK), jnp.int32),
                       pltpu.VMEM((2, CHUNK, 16), jnp.float32),
                       pltpu.VMEM((2, CHUNK, 16), jnp.float32),
                       pltpu.VMEM((2, CHUNK, HID), jnp.float32),
                       pltpu.SemaphoreType.DMA,
                       pltpu.SemaphoreType.DMA,
                       pltpu.SemaphoreType.DMA,
                       pltpu.SemaphoreType.DMA],
        compiler_params=pltpu.CompilerParams(use_tc_tiling_on_sc=False),
    )
    def gxk(x_hbm, dm_hbm, sm_hbm, xw_hbm,
            idx_d, idx_s, rows_d, rows_s, big, sd0, ss0, sd1, ss1):
        wid = lax.axis_index("s") * 2 + lax.axis_index("c")
        start = wid * ROWS_PER_W
        pltpu.sync_copy(dm_hbm.at[pl.ds(start, ROWS_PER_W)], idx_d)
        pltpu.sync_copy(sm_hbm.at[pl.ds(start, ROWS_PER_W)], idx_s)
        semd = (sd0, sd1)
        sems = (ss0, ss1)

        def issue(s, j):
            @pl.when((j < ROWS_PER_W) & (start + j < NROWS))
            def _():
                pltpu.async_copy(x_hbm.at[idx_d.at[j]], rows_d.at[s],
                                 semd[s])
                pltpu.async_copy(x_hbm.at[idx_s.at[j]], rows_s.at[s],
                                 sems[s])

        def flush(s, j):
            @pl.when((j < ROWS_PER_W) & (start + j < NROWS))
            def _():
                pltpu.make_async_copy(x_hbm.at[idx_d.at[j]], rows_d.at[s],
                                      semd[s]).wait()
                pltpu.make_async_copy(x_hbm.at[idx_s.at[j]], rows_s.at[s],
                                      sems[s]).wait()
                r = start + j
                pltpu.sync_copy(rows_d.at[s],
                                xw_hbm.at[pl.ds(r * CHUNK, CHUNK),
                                          pl.ds(0, 16)])
                pltpu.sync_copy(rows_s.at[s],
                                xw_hbm.at[pl.ds(r * CHUNK, CHUNK),
                                          pl.ds(16, 16)])

        issue(0, 0)

        @pl.loop(0, ROWS_PER_W // 2)
        def _(p):
            issue(1, 2 * p + 1)
            flush(0, 2 * p)
            issue(0, 2 * p + 2)
            flush(1, 2 * p + 1)

    return gxk(xpad, dst_m, src_m)


NSUB = N // 16


def _scatter_den(s1e, dst_m):
    zeros16 = jnp.zeros((NSUB, HID), jnp.float32)

    @functools.partial(
        pl.kernel,
        out_type=jax.ShapeDtypeStruct((2, N, HID), jnp.float32),
        mesh=_SC_MESH,
        scratch_types=[pltpu.VMEM((ROWS_PER_W, CHUNK), jnp.int32),
                       pltpu.VMEM((2, CHUNK, HID), jnp.float32),
                       pltpu.VMEM_SHARED((N, HID), jnp.float32),
                       pltpu.SemaphoreType.DMA,
                       pltpu.SemaphoreType.DMA],
        compiler_params=pltpu.CompilerParams(use_tc_tiling_on_sc=False),
    )
    def sk(e_hbm, dm_hbm, z_hbm, accd_hbm, idxb, bufe, shd, se0, se1):
        cid = lax.axis_index("c")
        sid = lax.axis_index("s")
        wid = sid * 2 + cid
        start = wid * ROWS_PER_W
        seme = (se0, se1)
        pltpu.sync_copy(dm_hbm.at[pl.ds(start, ROWS_PER_W)], idxb)
        pltpu.sync_copy(z_hbm, shd.at[pl.ds(sid * NSUB, NSUB)])
        plsc.subcore_barrier()

        def load(s, j):
            @pl.when((j < ROWS_PER_W) & (start + j < NROWS))
            def _():
                r = start + j
                pltpu.async_copy(e_hbm.at[pl.ds(r * CHUNK, CHUNK)],
                                 bufe.at[s], seme[s])

        def scat(s, j):
            @pl.when((j < ROWS_PER_W) & (start + j < NROWS))
            def _():
                r = start + j
                pltpu.make_async_copy(e_hbm.at[pl.ds(r * CHUNK, CHUNK)],
                                      bufe.at[s], seme[s]).wait()
                pltpu.sync_copy(bufe.at[s], shd.at[idxb.at[j]], add=True)

        load(0, 0)

        @pl.loop(0, ROWS_PER_W // 2)
        def _(p):
            load(1, 2 * p + 1)
            scat(0, 2 * p)
            load(0, 2 * p + 2)
            scat(1, 2 * p + 1)

        plsc.subcore_barrier()
        pltpu.sync_copy(shd.at[pl.ds(sid * NSUB, NSUB)],
                        accd_hbm.at[cid, pl.ds(sid * NSUB, NSUB)])

    return sk(s1e, dst_m, zeros16)


def _scatter2(s2, dst_m):
    zeros = jnp.zeros((NSUB, HID), jnp.float32)

    @functools.partial(
        pl.kernel,
        out_type=jax.ShapeDtypeStruct((2, N, HID), jnp.float32),
        mesh=_SC_MESH,
        scratch_types=[pltpu.VMEM((ROWS_PER_W, CHUNK), jnp.int32),
                       pltpu.VMEM((2, CHUNK, HID), jnp.float32),
                       pltpu.VMEM_SHARED((N, HID), jnp.float32),
                       pltpu.SemaphoreType.DMA,
                       pltpu.SemaphoreType.DMA],
        compiler_params=pltpu.CompilerParams(use_tc_tiling_on_sc=False),
    )
    def sk(rows_hbm, dm_hbm, z_hbm, acc_hbm, idxb, bufw, shw, sw0, sw1):
        cid = lax.axis_index("c")
        sid = lax.axis_index("s")
        wid = sid * 2 + cid
        start = wid * ROWS_PER_W
        semw = (sw0, sw1)
        pltpu.sync_copy(dm_hbm.at[pl.ds(start, ROWS_PER_W)], idxb)
        pltpu.sync_copy(z_hbm, shw.at[pl.ds(sid * NSUB, NSUB)])
        plsc.subcore_barrier()

        def load(s, j):
            @pl.when((j < ROWS_PER_W) & (start + j < NROWS))
            def _():
                r = start + j
                pltpu.async_copy(rows_hbm.at[pl.ds(r * CHUNK, CHUNK)],
                                 bufw.at[s], semw[s])

        def scat(s, j):
            @pl.when((j < ROWS_PER_W) & (start + j < NROWS))
            def _():
                r = start + j
                pltpu.make_async_copy(rows_hbm.at[pl.ds(r * CHUNK, CHUNK)],
                                      bufw.at[s], semw[s]).wait()
                pltpu.sync_copy(bufw.at[s], shw.at[idxb.at[j]], add=True)

        load(0, 0)

        @pl.loop(0, ROWS_PER_W // 2)
        def _(p):
            load(1, 2 * p + 1)
            scat(0, 2 * p)
            load(0, 2 * p + 2)
            scat(1, 2 * p + 1)

        plsc.subcore_barrier()
        pltpu.sync_copy(shw.at[pl.ds(sid * NSUB, NSUB)],
                        acc_hbm.at[cid, pl.ds(sid * NSUB, NSUB)])

    return sk(s2, dst_m, zeros)


def _prep_kv_mlp(p):
    w1 = p["W1"]
    return {
        "ea": w1[0:EDGE_DIM],
        "rf": w1[EDGE_DIM:EDGE_DIM + NG * EDGE_DIM],
        "hd": w1[84:84 + HID],
        "hs": w1[84 + HID:84 + 2 * HID],
        "iv": w1[84 + 2 * HID:],
        "b1": p["b1"].reshape(1, -1),
        "g": p["g"].reshape(1, -1),
        "be": p["be"].reshape(1, -1),
        "W2": p["W2"],
        "b2": p["b2"].reshape(1, -1),
    }


def _prep_q_mlp(p):
    return (p["W1"], p["b1"].reshape(1, -1), p["g"].reshape(1, -1),
            p["be"].reshape(1, -1), p["W2"], p["b2"].reshape(1, -1))


def kernel(h, x, edge_attr, edge_index, invar_ligand_shape, ligand_shape_emb,
           topo_out, e_w, params):
    del topo_out
    src = edge_index[0]
    dst = edge_index[1]
    dst_m = jnp.pad(dst.reshape(NROWS, CHUNK), ((0, NWORK * ROWS_PER_W - NROWS), (0, 0)))
    src_m = jnp.pad(src.reshape(NROWS, CHUNK), ((0, NWORK * ROWS_PER_W - NROWS), (0, 0)))
    dst_m64 = jnp.pad(dst.reshape(NCHTOT, CHUNK2), ((0, NWORK * NCH - NCHTOT), (0, 0)))
    src_m64 = jnp.pad(src.reshape(NCHTOT, CHUNK2), ((0, NWORK * NCH - NCHTOT), (0, 0)))
    ew = e_w.reshape(E, 1)
    xpad = jnp.pad(x, ((0, 0), (0, 13)))

    perm = np.array([(j % HEADS) * DH + j // HEADS for j in range(HID)],
                    dtype=np.int32)

    px = params["x2h"]
    hk = _prep_kv_mlp(px["hk"])
    hv = _prep_kv_mlp(px["hv"])
    hq = _prep_q_mlp(px["hq"])
    no = px["node_out"]
    n_w1 = jnp.concatenate([no["W1"][0:HID][perm], no["W1"][HID:]], axis=0)
    wn = (n_w1, no["b1"].reshape(1, -1), no["g"].reshape(1, -1),
          no["be"].reshape(1, -1), no["W2"], no["b2"].reshape(1, -1))

    ph = params["h2x"]
    xk = _prep_kv_mlp(ph["xk"])
    xv = _prep_kv_mlp(ph["xv"])
    xq = _prep_q_mlp(ph["xq"])
    wft = ph["Wf"].T
    wdt = ph["Wd"].T

    def kv_pack(m):
        return (m["hd"], m["iv"], m["b1"], m["hs"])

    xw = _gather_rel(xpad, dst_m, src_m)

    td1, ts1 = _node_tables(h, invar_ligand_shape,
                            kv_pack(hk), kv_pack(hv), hq)
    gd1, gs1 = _gather(td1, ts1, dst_m64, src_m64)
    hv_w2p = hv["W2"][:, perm]
    hv_b2p = hv["b2"][:, perm]
    s1w, s1d = _edge1(gd1, gs1, xw, edge_attr, ew,
                      (hk["ea"], hk["rf"], hk["g"], hk["be"],
                       hk["W2"], hk["b2"]),
                      (hv["ea"], hv["rf"], hv["g"], hv["be"],
                       hv_w2p, hv_b2p))
    accw1 = _scatter2(s1w, dst_m)
    accd1 = _scatter_den(s1d, dst_m)

    h_out, td2, ts2 = _node2(accw1, accd1, h, invar_ligand_shape, wn,
                             kv_pack(xk), kv_pack(xv), xq)

    gd2, gs2 = _gather(td2, ts2, dst_m64, src_m64)
    s2 = _edge2(gd2, gs2, xw, edge_attr, ew,
                (xk["ea"], xk["rf"], xk["g"], xk["be"], xk["W2"], xk["b2"]),
                (xv["ea"], xv["rf"], xv["g"], xv["be"], xv["W2"],
                 xv["b2"]))
    acc2 = _scatter2(s2, dst_m)

    se0 = ligand_shape_emb[:, :, 0]
    se1 = ligand_shape_emb[:, :, 1]
    se2 = ligand_shape_emb[:, :, 2]
    x_out = _tail(acc2, x, se0, se1, se2, wft, wdt)
    return h_out, x_out


# speedup vs baseline: 1.0194x; 1.0194x over previous
import functools

import numpy as np
import jax
import jax.numpy as jnp
from jax import lax
from jax.experimental import pallas as pl
from jax.experimental.pallas import tpu as pltpu
from jax.experimental.pallas import tpu_sc as plsc

N = 10000
E = 160000
HID = 128
HEADS = 16
DH = HID // HEADS
SHAPE_DIM = 16
EDGE_DIM = 4
NG = 20
R_MIN, R_MAX = 0.0, 10.0
RSQRT_DH = float(1.0 / np.sqrt(DH))
STEP = (R_MAX - R_MIN) / (NG - 1)
COEFF = -0.5 / STEP**2

NODE_BLK = 1000
EDGE_BLK = 1280

_INTERPRET = False

TD_W = 384
TS_W = 256


def _ln_mxu(hdn, g, be):
    o1 = jnp.full((HID, HEADS), 1.0 / HID, jnp.float32)
    o2 = jnp.full((HEADS, HID), 1.0 / HEADS, jnp.float32)
    mu = (hdn @ o1) @ o2
    ex2 = ((hdn * hdn) @ o1) @ o2
    var = ex2 - mu * mu
    return (hdn - mu) * jax.lax.rsqrt(var + 1e-5) * g + be


def _sel(rows, cols, fn):
    r = jax.lax.broadcasted_iota(jnp.int32, (rows, cols), 0)
    c = jax.lax.broadcasted_iota(jnp.int32, (rows, cols), 1)
    return fn(r, c).astype(jnp.float32)


def _full_spec(a):
    nd = a.ndim
    return pl.BlockSpec(a.shape, lambda i, *, _nd=nd: (0,) * _nd)


def _head_sum_matrix():
    j = jax.lax.broadcasted_iota(jnp.int32, (HID, HEADS), 0)
    h = jax.lax.broadcasted_iota(jnp.int32, (HID, HEADS), 1)
    return (j // DH == h).astype(jnp.float32)


def _tables_math(h, inv, wk, wv, wq):
    A_k = h @ wk[0] + inv @ wk[1] + wk[2]
    A_v = h @ wv[0] + inv @ wv[1] + wv[2]
    B_k = h @ wk[3]
    B_v = h @ wv[3]
    hdnq = h @ wq[0] + wq[1]
    q = jnp.maximum(_ln_mxu(hdnq, wq[2], wq[3]), 0.0) @ wq[4] + wq[5]
    td = jnp.concatenate([A_k, A_v, q], axis=1)
    ts = jnp.concatenate([B_k, B_v], axis=1)
    return td, ts


def _node1_body(h_ref, inv_ref, *rest):
    (wk0, wk1, wk2, wk3, wv0, wv1, wv2, wv3,
     q0, q1, q2, q3, q4, q5, td_ref, ts_ref) = rest
    td, ts = _tables_math(
        h_ref[...], inv_ref[...],
        (wk0[...], wk1[...], wk2[...], wk3[...]),
        (wv0[...], wv1[...], wv2[...], wv3[...]),
        (q0[...], q1[...], q2[...], q3[...], q4[...], q5[...]))
    td_ref[...] = td
    ts_ref[...] = ts


def _node_tables(h, inv, wk, wv, wq):
    args = [h, inv, *wk, *wv, *wq]
    in_specs = [pl.BlockSpec((NODE_BLK, h.shape[1]), lambda i: (i, 0)),
                pl.BlockSpec((NODE_BLK, SHAPE_DIM), lambda i: (i, 0))]
    in_specs += [_full_spec(a) for a in args[2:]]
    return pl.pallas_call(
        _node1_body,
        grid=(N // NODE_BLK,),
        in_specs=in_specs,
        out_specs=[pl.BlockSpec((NODE_BLK, TD_W), lambda i: (i, 0)),
                   pl.BlockSpec((NODE_BLK, TS_W), lambda i: (i, 0))],
        out_shape=[jax.ShapeDtypeStruct((N, TD_W), jnp.float32),
                   jax.ShapeDtypeStruct((N, TS_W), jnp.float32)],
        interpret=_INTERPRET,
    )(*args)


def _edge_feats(xw, ea):
    rel = xw[:, 0:16] - xw[:, 16:32]
    d2 = (rel * rel) @ jnp.ones((16, NG), jnp.float32)
    dist = jnp.sqrt(d2 + 1e-12)
    offs = jax.lax.broadcasted_iota(jnp.int32, (1, NG), 1).astype(
        jnp.float32) * STEP
    df = jnp.exp(COEFF * (dist - offs) ** 2)
    ea_b = ea @ _sel(EDGE_DIM, NG * EDGE_DIM, lambda r, c: c // NG == r)
    df_b = df @ _sel(NG, NG * EDGE_DIM, lambda r, c: c % NG == r)
    return ea_b * df_b, rel


def _edge_mlp(ea, rf, gd, gs, off, w1a, w1r, g, be, w2, b2):
    hdn = ea @ w1a + rf @ w1r + gd[:, off:off + HID] + gs[:, off:off + HID]
    return jnp.maximum(_ln_mxu(hdn, g, be), 0.0) @ w2 + b2


def _edge1_body(gd_ref, gs_ref, xw_ref, ea_ref, ew_ref, *rest):
    (k_w1a, k_w1r, k_g, k_be, k_w2, k_b2,
     v_w1a, v_w1r, v_g, v_be, v_w2p, v_b2p, s1w_ref, s1d_ref) = rest
    gd = gd_ref[...]
    gs = gs_ref[...]
    ea = ea_ref[...]
    rf, _ = _edge_feats(xw_ref[...], ea)
    kk = _edge_mlp(ea, rf, gd, gs, 0, k_w1a[...], k_w1r[...], k_g[...],
                   k_be[...], k_w2[...], k_b2[...])
    q = gd[:, 256:384]
    logits = ((q * kk) @ _head_sum_matrix()) * RSQRT_DH
    ex = jnp.exp(logits)
    vt = _edge_mlp(ea, rf, gd, gs, HID, v_w1a[...], v_w1r[...], v_g[...],
                   v_be[...], v_w2p[...], v_b2p[...])
    ext = ex @ _sel(HEADS, HID, lambda r, c: c % HEADS == r)
    extw = ext * (ew_ref[...] @ jnp.ones((1, HID), jnp.float32))
    s1w_ref[...] = extw * vt
    s1d_ref[...] = jnp.concatenate(
        [ex, jnp.zeros((ex.shape[0], HID - HEADS), jnp.float32)], axis=1)


def _edge1(gd, gs, xw, ea, ew, wk, wv):
    args = [gd, gs, xw, ea, ew, *wk, *wv]
    in_specs = [pl.BlockSpec((EDGE_BLK, TD_W), lambda i: (i, 0)),
                pl.BlockSpec((EDGE_BLK, TS_W), lambda i: (i, 0)),
                pl.BlockSpec((EDGE_BLK, HID), lambda i: (i, 0)),
                pl.BlockSpec((EDGE_BLK, EDGE_DIM), lambda i: (i, 0)),
                pl.BlockSpec((EDGE_BLK, 1), lambda i: (i, 0))]
    in_specs += [_full_spec(a) for a in args[5:]]
    return pl.pallas_call(
        _edge1_body,
        grid=(E // EDGE_BLK,),
        in_specs=in_specs,
        out_specs=[pl.BlockSpec((EDGE_BLK, HID), lambda i: (i, 0)),
                   pl.BlockSpec((EDGE_BLK, HID), lambda i: (i, 0))],
        out_shape=[jax.ShapeDtypeStruct((E, HID), jnp.float32),
                   jax.ShapeDtypeStruct((E, HID), jnp.float32)],
        interpret=_INTERPRET,
    )(*args)


def _edge2_body(gd_ref, gs_ref, xw_ref, ea_ref, ew_ref, *rest):
    (k_w1a, k_w1r, k_g, k_be, k_w2, k_b2,
     v_w1a, v_w1r, v_g, v_be, v_w2, v_b2, s2_ref) = rest
    gd = gd_ref[...]
    gs = gs_ref[...]
    ea = ea_ref[...]
    rf, rel = _edge_feats(xw_ref[...], ea)
    kk = _edge_mlp(ea, rf, gd, gs, 0, k_w1a[...], k_w1r[...], k_g[...],
                   k_be[...], k_w2[...], k_b2[...])
    q = gd[:, 256:384]
    logits = ((q * kk) @ _head_sum_matrix()) * RSQRT_DH
    ex = jnp.exp(logits)
    v2 = _edge_mlp(ea, rf, gd, gs, HID, v_w1a[...], v_w1r[...], v_g[...],
                   v_be[...], v_w2[...], v_b2[...])
    vv = ex * (ew_ref[...] @ jnp.ones((1, HEADS), jnp.float32)) * v2
    vv3 = vv @ _sel(HEADS, 48, lambda r, c: c % HEADS == r)
    rel3 = rel @ _sel(16, 48, lambda r, c: c // HEADS == r)
    zpad = jnp.zeros((vv3.shape[0], 64), jnp.float32)
    s2_ref[...] = jnp.concatenate([vv3 * rel3, ex, zpad], axis=1)


def _edge2(gd, gs, xw, ea, ew, wk, wv):
    args = [gd, gs, xw, ea, ew, *wk, *wv]
    in_specs = [pl.BlockSpec((EDGE_BLK, TD_W), lambda i: (i, 0)),
                pl.BlockSpec((EDGE_BLK, TS_W), lambda i: (i, 0)),
                pl.BlockSpec((EDGE_BLK, HID), lambda i: (i, 0)),
                pl.BlockSpec((EDGE_BLK, EDGE_DIM), lambda i: (i, 0)),
                pl.BlockSpec((EDGE_BLK, 1), lambda i: (i, 0))]
    in_specs += [_full_spec(a) for a in args[5:]]
    return pl.pallas_call(
        _edge2_body,
        grid=(E // EDGE_BLK,),
        in_specs=in_specs,
        out_specs=pl.BlockSpec((EDGE_BLK, HID), lambda i: (i, 0)),
        out_shape=jax.ShapeDtypeStruct((E, HID), jnp.float32),
        interpret=_INTERPRET,
    )(*args)


def _node2_body(accw_ref, accd_ref, h_ref, inv_ref, *rest):
    (n_w1, n_b1, n_g, n_be, n_w2, n_b2,
     wk0, wk1, wk2, wk3, wv0, wv1, wv2, wv3,
     q0, q1, q2, q3, q4, q5, ho_ref, td_ref, ts_ref) = rest
    accw = accw_ref[...]
    h = h_ref[...]
    num = accw[0] + accw[1]
    accd = accd_ref[...]
    dent = (accd[0] + accd[1]) @ _sel(HEADS, HID, lambda r, c: c % HEADS == r)
    out_t = num / (dent + 1e-16)
    u = jnp.concatenate([out_t, h], axis=1)
    hdn = u @ n_w1[...] + n_b1[...]
    ho = jnp.maximum(_ln_mxu(hdn, n_g[...], n_be[...]), 0.0) @ n_w2[...] \
        + n_b2[...] + h
    ho_ref[...] = ho
    td, ts = _tables_math(
        ho, inv_ref[...],
        (wk0[...], wk1[...], wk2[...], wk3[...]),
        (wv0[...], wv1[...], wv2[...], wv3[...]),
        (q0[...], q1[...], q2[...], q3[...], q4[...], q5[...]))
    td_ref[...] = td
    ts_ref[...] = ts


def _node2(accw, accd, h, inv, wn, wk, wv, wq):
    args = [accw, accd, h, inv, *wn, *wk, *wv, *wq]
    in_specs = [pl.BlockSpec((2, NODE_BLK, HID), lambda i: (0, i, 0)),
                pl.BlockSpec((2, NODE_BLK, HEADS), lambda i: (0, i, 0)),
                pl.BlockSpec((NODE_BLK, HID), lambda i: (i, 0)),
                pl.BlockSpec((NODE_BLK, SHAPE_DIM), lambda i: (i, 0))]
    in_specs += [_full_spec(a) for a in args[4:]]
    return pl.pallas_call(
        _node2_body,
        grid=(N // NODE_BLK,),
        in_specs=in_specs,
        out_specs=[pl.BlockSpec((NODE_BLK, HID), lambda i: (i, 0)),
                   pl.BlockSpec((NODE_BLK, TD_W), lambda i: (i, 0)),
                   pl.BlockSpec((NODE_BLK, TS_W), lambda i: (i, 0))],
        out_shape=[jax.ShapeDtypeStruct((N, HID), jnp.float32),
                   jax.ShapeDtypeStruct((N, TD_W), jnp.float32),
                   jax.ShapeDtypeStruct((N, TS_W), jnp.float32)],
        interpret=_INTERPRET,
    )(*args)


def _tail_body(acc_ref, x_ref, se0_ref, se1_ref, se2_ref, wft_ref, wdt_ref,
               xo_ref):
    acc = acc_ref[...]
    x = x_ref[...]
    se = (se0_ref[...], se1_ref[...], se2_ref[...])
    den = acc[0, :, 48:64] + acc[1, :, 48:64]
    wft = wft_ref[...]
    wdt = wdt_ref[...]
    outs, Ps, Ds = [], [], []
    for c in range(3):
        num = acc[0, :, c * 16:(c + 1) * 16] + acc[1, :, c * 16:(c + 1) * 16]
        oc = num / (den + 1e-16)
        outs.append(oc)
        tmp = jnp.concatenate([x[:, c:c + 1], oc, se[c]], axis=1)
        Ps.append(tmp @ wft)
        Ds.append(tmp @ wdt)
    dot = Ps[0] * Ds[0] + Ps[1] * Ds[1] + Ps[2] * Ds[2]
    dsq = Ds[0] * Ds[0] + Ds[1] * Ds[1] + Ds[2] * Ds[2]
    coef = dot / (dsq + 1e-6)
    mask = dot >= 0.0
    deltas = []
    for c in range(3):
        neg = jnp.where(mask, Ps[c], Ps[c] - coef * Ds[c])
        res = 0.2 * Ps[c] + 0.8 * neg
        delta = jnp.mean(outs[c], axis=-1, keepdims=True) \
            + jnp.mean(res, axis=-1, keepdims=True)
        deltas.append(x[:, c:c + 1] + delta)
    xo_ref[...] = jnp.concatenate(deltas, axis=1)


def _tail(acc2, x, se0, se1, se2, wft, wdt):
    args = [acc2, x, se0, se1, se2, wft, wdt]
    in_specs = [pl.BlockSpec((2, NODE_BLK, HID), lambda i: (0, i, 0)),
                pl.BlockSpec((NODE_BLK, 3), lambda i: (i, 0)),
                pl.BlockSpec((NODE_BLK, 16), lambda i: (i, 0)),
                pl.BlockSpec((NODE_BLK, 16), lambda i: (i, 0)),
                pl.BlockSpec((NODE_BLK, 16), lambda i: (i, 0)),
                _full_spec(wft), _full_spec(wdt)]
    return pl.pallas_call(
        _tail_body,
        grid=(N // NODE_BLK,),
        in_specs=in_specs,
        out_specs=pl.BlockSpec((NODE_BLK, 3), lambda i: (i, 0)),
        out_shape=jax.ShapeDtypeStruct((N, 3), jnp.float32),
        interpret=_INTERPRET,
    )(*args)


CHUNK = 128
NROWS = E // CHUNK
NWORK = 32
ROWS_PER_W = -(-NROWS // NWORK)
NODES_PER_SUB = N // 16

_SC_MESH = plsc.VectorSubcoreMesh(core_axis_name="c", subcore_axis_name="s")


CHUNK2 = 64
NCHTOT = E // CHUNK2
NCH = -(-NCHTOT // NWORK) if NCHTOT % NWORK else NCHTOT // NWORK
NCH = 80


def _gather(td, ts, dst_m64, src_m64):
    @functools.partial(
        pl.kernel,
        out_type=[jax.ShapeDtypeStruct((E, TD_W), jnp.float32),
                  jax.ShapeDtypeStruct((E, TS_W), jnp.float32)],
        mesh=_SC_MESH,
        scratch_types=[pltpu.VMEM((NCH, CHUNK2), jnp.int32),
                       pltpu.VMEM((NCH, CHUNK2), jnp.int32),
                       pltpu.VMEM((2, CHUNK2, TD_W), jnp.float32),
                       pltpu.VMEM((2, CHUNK2, TS_W), jnp.float32),
                       pltpu.SemaphoreType.DMA,
                       pltpu.SemaphoreType.DMA,
                       pltpu.SemaphoreType.DMA,
                       pltpu.SemaphoreType.DMA],
    )
    def gk(td_hbm, ts_hbm, dm_hbm, sm_hbm, gd_hbm, gs_hbm,
           idx_d, idx_s, rows_d, rows_s, sd0, ss0, sd1, ss1):
        wid = lax.axis_index("s") * 2 + lax.axis_index("c")
        start = wid * NCH
        pltpu.sync_copy(dm_hbm.at[pl.ds(start, NCH)], idx_d)
        pltpu.sync_copy(sm_hbm.at[pl.ds(start, NCH)], idx_s)
        semd = (sd0, sd1)
        sems = (ss0, ss1)

        def issue(s, j):
            @pl.when((j < NCH) & (start + j < NCHTOT))
            def _():
                pltpu.async_copy(td_hbm.at[idx_d.at[j]], rows_d.at[s],
                                 semd[s])
                pltpu.async_copy(ts_hbm.at[idx_s.at[j]], rows_s.at[s],
                                 sems[s])

        def flush(s, j):
            @pl.when((j < NCH) & (start + j < NCHTOT))
            def _():
                pltpu.make_async_copy(td_hbm.at[idx_d.at[j]], rows_d.at[s],
                                      semd[s]).wait()
                pltpu.make_async_copy(ts_hbm.at[idx_s.at[j]], rows_s.at[s],
                                      sems[s]).wait()
                r = start + j
                pltpu.sync_copy(rows_d.at[s],
                                gd_hbm.at[pl.ds(r * CHUNK2, CHUNK2)])
                pltpu.sync_copy(rows_s.at[s],
                                gs_hbm.at[pl.ds(r * CHUNK2, CHUNK2)])

        issue(0, 0)

        @pl.loop(0, NCH // 2)
        def _(p):
            issue(1, 2 * p + 1)
            flush(0, 2 * p)
            issue(0, 2 * p + 2)
            flush(1, 2 * p + 1)

    return gk(td, ts, dst_m64, src_m64)


PCHUNK = CHUNK // 8


def _gather_rel(xpad, dst_m, src_m):
    @functools.partial(
        pl.kernel,
        out_type=jax.ShapeDtypeStruct((E, HID), jnp.float32),
        mesh=_SC_MESH,
        scratch_types=[pltpu.VMEM((ROWS_PER_W, CHUNK), jnp.int32),
                       pltpu.VMEM((ROWS_PER_W, CHUNK), jnp.int32),
                       pltpu.VMEM((2, CHUNK, 16), jnp.float32),
                       pltpu.VMEM((2, CHUNK, 16), jnp.float32),
                       pltpu.VMEM((2, CHUNK, HID), jnp.float32),
                       pltpu.SemaphoreType.DMA,
                       pltpu.SemaphoreType.DMA,
                       pltpu.SemaphoreType.DMA,
                       pltpu.SemaphoreType.DMA],
        compiler_params=pltpu.CompilerParams(use_tc_tiling_on_sc=False),
    )
    def gxk(x_hbm, dm_hbm, sm_hbm, xw_hbm,
            idx_d, idx_s, rows_d, rows_s, big, sd0, ss0, sd1, ss1):
        wid = lax.axis_index("s") * 2 + lax.axis_index("c")
        start = wid * ROWS_PER_W
        pltpu.sync_copy(dm_hbm.at[pl.ds(start, ROWS_PER_W)], idx_d)
        pltpu.sync_copy(sm_hbm.at[pl.ds(start, ROWS_PER_W)], idx_s)
        semd = (sd0, sd1)
        sems = (ss0, ss1)

        def issue(s, j):
            @pl.when((j < ROWS_PER_W) & (start + j < NROWS))
            def _():
                pltpu.async_copy(x_hbm.at[idx_d.at[j]], rows_d.at[s],
                                 semd[s])
                pltpu.async_copy(x_hbm.at[idx_s.at[j]], rows_s.at[s],
                                 sems[s])

        def flush(s, j):
            @pl.when((j < ROWS_PER_W) & (start + j < NROWS))
            def _():
                pltpu.make_async_copy(x_hbm.at[idx_d.at[j]], rows_d.at[s],
                                      semd[s]).wait()
                pltpu.make_async_copy(x_hbm.at[idx_s.at[j]], rows_s.at[s],
                                      sems[s]).wait()
                r = start + j
                pltpu.sync_copy(rows_d.at[s],
                                xw_hbm.at[pl.ds(r * CHUNK, CHUNK),
                                          pl.ds(0, 16)])
                pltpu.sync_copy(rows_s.at[s],
                                xw_hbm.at[pl.ds(r * CHUNK, CHUNK),
                                          pl.ds(16, 16)])

        issue(0, 0)

        @pl.loop(0, ROWS_PER_W // 2)
        def _(p):
            issue(1, 2 * p + 1)
            flush(0, 2 * p)
            issue(0, 2 * p + 2)
            flush(1, 2 * p + 1)

    return gxk(xpad, dst_m, src_m)


NSUB = N // 16


def _scatter_den(s1e, dst_m):
    zeros16 = jnp.zeros((NSUB, HEADS), jnp.float32)

    @functools.partial(
        pl.kernel,
        out_type=jax.ShapeDtypeStruct((2, N, HEADS), jnp.float32),
        mesh=_SC_MESH,
        scratch_types=[pltpu.VMEM((ROWS_PER_W, CHUNK), jnp.int32),
                       pltpu.VMEM((2, CHUNK, HEADS), jnp.float32),
                       pltpu.VMEM_SHARED((N, HEADS), jnp.float32),
                       pltpu.SemaphoreType.DMA,
                       pltpu.SemaphoreType.DMA],
        compiler_params=pltpu.CompilerParams(use_tc_tiling_on_sc=False),
    )
    def sk(e_hbm, dm_hbm, z_hbm, accd_hbm, idxb, bufe, shd, se0, se1):
        cid = lax.axis_index("c")
        sid = lax.axis_index("s")
        wid = sid * 2 + cid
        start = wid * ROWS_PER_W
        seme = (se0, se1)
        pltpu.sync_copy(dm_hbm.at[pl.ds(start, ROWS_PER_W)], idxb)
        pltpu.sync_copy(z_hbm, shd.at[pl.ds(sid * NSUB, NSUB)])
        plsc.subcore_barrier()

        def load(s, j):
            @pl.when((j < ROWS_PER_W) & (start + j < NROWS))
            def _():
                r = start + j
                pltpu.async_copy(e_hbm.at[pl.ds(r * CHUNK, CHUNK),
                                          pl.ds(0, HEADS)],
                                 bufe.at[s], seme[s])

        def scat(s, j):
            @pl.when((j < ROWS_PER_W) & (start + j < NROWS))
            def _():
                r = start + j
                pltpu.make_async_copy(e_hbm.at[pl.ds(r * CHUNK, CHUNK),
                                               pl.ds(0, HEADS)],
                                      bufe.at[s], seme[s]).wait()
                pltpu.sync_copy(bufe.at[s], shd.at[idxb.at[j]], add=True)

        load(0, 0)

        @pl.loop(0, ROWS_PER_W // 2)
        def _(p):
            load(1, 2 * p + 1)
            scat(0, 2 * p)
            load(0, 2 * p + 2)
            scat(1, 2 * p + 1)

        plsc.subcore_barrier()
        pltpu.sync_copy(shd.at[pl.ds(sid * NSUB, NSUB)],
                        accd_hbm.at[cid, pl.ds(sid * NSUB, NSUB)])

    return sk(s1e, dst_m, zeros16)


def _scatter2(s2, dst_m):
    zeros = jnp.zeros((NSUB, HID), jnp.float32)

    @functools.partial(
        pl.kernel,
        out_type=jax.ShapeDtypeStruct((2, N, HID), jnp.float32),
        mesh=_SC_MESH,
        scratch_types=[pltpu.VMEM((ROWS_PER_W, CHUNK), jnp.int32),
                       pltpu.VMEM((2, CHUNK, HID), jnp.float32),
                       pltpu.VMEM_SHARED((N, HID), jnp.float32),
                       pltpu.SemaphoreType.DMA,
                       pltpu.SemaphoreType.DMA],
        compiler_params=pltpu.CompilerParams(use_tc_tiling_on_sc=False),
    )
    def sk(rows_hbm, dm_hbm, z_hbm, acc_hbm, idxb, bufw, shw, sw0, sw1):
        cid = lax.axis_index("c")
        sid = lax.axis_index("s")
        wid = sid * 2 + cid
        start = wid * ROWS_PER_W
        semw = (sw0, sw1)
        pltpu.sync_copy(dm_hbm.at[pl.ds(start, ROWS_PER_W)], idxb)
        pltpu.sync_copy(z_hbm, shw.at[pl.ds(sid * NSUB, NSUB)])
        plsc.subcore_barrier()

        def load(s, j):
            @pl.when((j < ROWS_PER_W) & (start + j < NROWS))
            def _():
                r = start + j
                pltpu.async_copy(rows_hbm.at[pl.ds(r * CHUNK, CHUNK)],
                                 bufw.at[s], semw[s])

        def scat(s, j):
            @pl.when((j < ROWS_PER_W) & (start + j < NROWS))
            def _():
                r = start + j
                pltpu.make_async_copy(rows_hbm.at[pl.ds(r * CHUNK, CHUNK)],
                                      bufw.at[s], semw[s]).wait()
                pltpu.sync_copy(bufw.at[s], shw.at[idxb.at[j]], add=True)

        load(0, 0)

        @pl.loop(0, ROWS_PER_W // 2)
        def _(p):
            load(1, 2 * p + 1)
            scat(0, 2 * p)
            load(0, 2 * p + 2)
            scat(1, 2 * p + 1)

        plsc.subcore_barrier()
        pltpu.sync_copy(shw.at[pl.ds(sid * NSUB, NSUB)],
                        acc_hbm.at[cid, pl.ds(sid * NSUB, NSUB)])

    return sk(s2, dst_m, zeros)


def _prep_kv_mlp(p):
    w1 = p["W1"]
    return {
        "ea": w1[0:EDGE_DIM],
        "rf": w1[EDGE_DIM:EDGE_DIM + NG * EDGE_DIM],
        "hd": w1[84:84 + HID],
        "hs": w1[84 + HID:84 + 2 * HID],
        "iv": w1[84 + 2 * HID:],
        "b1": p["b1"].reshape(1, -1),
        "g": p["g"].reshape(1, -1),
        "be": p["be"].reshape(1, -1),
        "W2": p["W2"],
        "b2": p["b2"].reshape(1, -1),
    }


def _prep_q_mlp(p):
    return (p["W1"], p["b1"].reshape(1, -1), p["g"].reshape(1, -1),
            p["be"].reshape(1, -1), p["W2"], p["b2"].reshape(1, -1))


def kernel(h, x, edge_attr, edge_index, invar_ligand_shape, ligand_shape_emb,
           topo_out, e_w, params):
    del topo_out
    src = edge_index[0]
    dst = edge_index[1]
    dst_m = jnp.pad(dst.reshape(NROWS, CHUNK), ((0, NWORK * ROWS_PER_W - NROWS), (0, 0)))
    src_m = jnp.pad(src.reshape(NROWS, CHUNK), ((0, NWORK * ROWS_PER_W - NROWS), (0, 0)))
    dst_m64 = jnp.pad(dst.reshape(NCHTOT, CHUNK2), ((0, NWORK * NCH - NCHTOT), (0, 0)))
    src_m64 = jnp.pad(src.reshape(NCHTOT, CHUNK2), ((0, NWORK * NCH - NCHTOT), (0, 0)))
    ew = e_w.reshape(E, 1)
    xpad = jnp.pad(x, ((0, 0), (0, 13)))

    perm = np.array([(j % HEADS) * DH + j // HEADS for j in range(HID)],
                    dtype=np.int32)

    px = params["x2h"]
    hk = _prep_kv_mlp(px["hk"])
    hv = _prep_kv_mlp(px["hv"])
    hq = _prep_q_mlp(px["hq"])
    no = px["node_out"]
    n_w1 = jnp.concatenate([no["W1"][0:HID][perm], no["W1"][HID:]], axis=0)
    wn = (n_w1, no["b1"].reshape(1, -1), no["g"].reshape(1, -1),
          no["be"].reshape(1, -1), no["W2"], no["b2"].reshape(1, -1))

    ph = params["h2x"]
    xk = _prep_kv_mlp(ph["xk"])
    xv = _prep_kv_mlp(ph["xv"])
    xq = _prep_q_mlp(ph["xq"])
    wft = ph["Wf"].T
    wdt = ph["Wd"].T

    def kv_pack(m):
        return (m["hd"], m["iv"], m["b1"], m["hs"])

    xw = _gather_rel(xpad, dst_m, src_m)

    td1, ts1 = _node_tables(h, invar_ligand_shape,
                            kv_pack(hk), kv_pack(hv), hq)
    gd1, gs1 = _gather(td1, ts1, dst_m64, src_m64)
    hv_w2p = hv["W2"][:, perm]
    hv_b2p = hv["b2"][:, perm]
    s1w, s1d = _edge1(gd1, gs1, xw, edge_attr, ew,
                      (hk["ea"], hk["rf"], hk["g"], hk["be"],
                       hk["W2"], hk["b2"]),
                      (hv["ea"], hv["rf"], hv["g"], hv["be"],
                       hv_w2p, hv_b2p))
    accw1 = _scatter2(s1w, dst_m)
    accd1 = _scatter_den(s1d, dst_m)

    h_out, td2, ts2 = _node2(accw1, accd1, h, invar_ligand_shape, wn,
                             kv_pack(xk), kv_pack(xv), xq)

    gd2, gs2 = _gather(td2, ts2, dst_m64, src_m64)
    s2 = _edge2(gd2, gs2, xw, edge_attr, ew,
                (xk["ea"], xk["rf"], xk["g"], xk["be"], xk["W2"], xk["b2"]),
                (xv["ea"], xv["rf"], xv["g"], xv["be"], xv["W2"],
                 xv["b2"]))
    acc2 = _scatter2(s2, dst_m)

    se0 = ligand_shape_emb[:, :, 0]
    se1 = ligand_shape_emb[:, :, 1]
    se2 = ligand_shape_emb[:, :, 2]
    x_out = _tail(acc2, x, se0, se1, se2, wft, wdt)
    return h_out, x_out


# speedup vs baseline: 1.0875x; 1.0668x over previous
import functools

import numpy as np
import jax
import jax.numpy as jnp
from jax import lax
from jax.experimental import pallas as pl
from jax.experimental.pallas import tpu as pltpu
from jax.experimental.pallas import tpu_sc as plsc

N = 10000
E = 160000
HID = 128
HEADS = 16
DH = HID // HEADS
SHAPE_DIM = 16
EDGE_DIM = 4
NG = 20
R_MIN, R_MAX = 0.0, 10.0
RSQRT_DH = float(1.0 / np.sqrt(DH))
STEP = (R_MAX - R_MIN) / (NG - 1)
COEFF = -0.5 / STEP**2

NODE_BLK = 1000
EDGE_BLK = 2560

_INTERPRET = False

TD_W = 384
TS_W = 256


def _ln_mxu(hdn, g, be):
    o1 = jnp.full((HID, HEADS), 1.0 / HID, jnp.float32)
    o2 = jnp.full((HEADS, HID), 1.0 / HEADS, jnp.float32)
    mu = (hdn @ o1) @ o2
    ex2 = ((hdn * hdn) @ o1) @ o2
    var = ex2 - mu * mu
    return (hdn - mu) * jax.lax.rsqrt(var + 1e-5) * g + be


def _sel(rows, cols, fn):
    r = jax.lax.broadcasted_iota(jnp.int32, (rows, cols), 0)
    c = jax.lax.broadcasted_iota(jnp.int32, (rows, cols), 1)
    return fn(r, c).astype(jnp.float32)


def _full_spec(a):
    nd = a.ndim
    return pl.BlockSpec(a.shape, lambda i, *, _nd=nd: (0,) * _nd)


def _head_sum_matrix():
    j = jax.lax.broadcasted_iota(jnp.int32, (HID, HEADS), 0)
    h = jax.lax.broadcasted_iota(jnp.int32, (HID, HEADS), 1)
    return (j // DH == h).astype(jnp.float32)


def _tables_math(h, inv, wk, wv, wq):
    A_k = h @ wk[0] + inv @ wk[1] + wk[2]
    A_v = h @ wv[0] + inv @ wv[1] + wv[2]
    B_k = h @ wk[3]
    B_v = h @ wv[3]
    hdnq = h @ wq[0] + wq[1]
    q = jnp.maximum(_ln_mxu(hdnq, wq[2], wq[3]), 0.0) @ wq[4] + wq[5]
    td = jnp.concatenate([A_k, A_v, q], axis=1)
    ts = jnp.concatenate([B_k, B_v], axis=1)
    return td, ts


def _node1_body(h_ref, inv_ref, *rest):
    (wk0, wk1, wk2, wk3, wv0, wv1, wv2, wv3,
     q0, q1, q2, q3, q4, q5, td_ref, ts_ref) = rest
    td, ts = _tables_math(
        h_ref[...], inv_ref[...],
        (wk0[...], wk1[...], wk2[...], wk3[...]),
        (wv0[...], wv1[...], wv2[...], wv3[...]),
        (q0[...], q1[...], q2[...], q3[...], q4[...], q5[...]))
    td_ref[...] = td
    ts_ref[...] = ts


def _node_tables(h, inv, wk, wv, wq):
    args = [h, inv, *wk, *wv, *wq]
    in_specs = [pl.BlockSpec((NODE_BLK, h.shape[1]), lambda i: (i, 0)),
                pl.BlockSpec((NODE_BLK, SHAPE_DIM), lambda i: (i, 0))]
    in_specs += [_full_spec(a) for a in args[2:]]
    return pl.pallas_call(
        _node1_body,
        grid=(N // NODE_BLK,),
        in_specs=in_specs,
        out_specs=[pl.BlockSpec((NODE_BLK, TD_W), lambda i: (i, 0)),
                   pl.BlockSpec((NODE_BLK, TS_W), lambda i: (i, 0))],
        out_shape=[jax.ShapeDtypeStruct((N, TD_W), jnp.float32),
                   jax.ShapeDtypeStruct((N, TS_W), jnp.float32)],
        interpret=_INTERPRET,
    )(*args)


def _edge_feats(xw, ea):
    rel = xw[:, 0:16] - xw[:, 16:32]
    d2 = (rel * rel) @ jnp.ones((16, NG), jnp.float32)
    dist = jnp.sqrt(d2 + 1e-12)
    offs = jax.lax.broadcasted_iota(jnp.int32, (1, NG), 1).astype(
        jnp.float32) * STEP
    df = jnp.exp(COEFF * (dist - offs) ** 2)
    ea_b = ea @ _sel(EDGE_DIM, NG * EDGE_DIM, lambda r, c: c // NG == r)
    df_b = df @ _sel(NG, NG * EDGE_DIM, lambda r, c: c % NG == r)
    return ea_b * df_b, rel


def _edge_mlp(ea, rf, gd, gs, off, w1a, w1r, g, be, w2, b2):
    hdn = ea @ w1a + rf @ w1r + gd[:, off:off + HID] + gs[:, off:off + HID]
    return jnp.maximum(_ln_mxu(hdn, g, be), 0.0) @ w2 + b2


def _edge1_body(gd_ref, gs_ref, xw_ref, ea_ref, ew_ref, *rest):
    (k_w1a, k_w1r, k_g, k_be, k_w2, k_b2,
     v_w1a, v_w1r, v_g, v_be, v_w2p, v_b2p, s1w_ref, s1d_ref) = rest
    gd = gd_ref[...]
    gs = gs_ref[...]
    ea = ea_ref[...]
    rf, _ = _edge_feats(xw_ref[...], ea)
    kk = _edge_mlp(ea, rf, gd, gs, 0, k_w1a[...], k_w1r[...], k_g[...],
                   k_be[...], k_w2[...], k_b2[...])
    q = gd[:, 256:384]
    logits = ((q * kk) @ _head_sum_matrix()) * RSQRT_DH
    ex = jnp.exp(logits)
    vt = _edge_mlp(ea, rf, gd, gs, HID, v_w1a[...], v_w1r[...], v_g[...],
                   v_be[...], v_w2p[...], v_b2p[...])
    ext = ex @ _sel(HEADS, HID, lambda r, c: c % HEADS == r)
    extw = ext * (ew_ref[...] @ jnp.ones((1, HID), jnp.float32))
    s1w_ref[...] = extw * vt
    s1d_ref[...] = jnp.concatenate(
        [ex, jnp.zeros((ex.shape[0], HID - HEADS), jnp.float32)], axis=1)


def _edge1(gd, gs, xw, ea, ew, wk, wv):
    args = [gd, gs, xw, ea, ew, *wk, *wv]
    in_specs = [pl.BlockSpec((EDGE_BLK, TD_W), lambda i: (i, 0)),
                pl.BlockSpec((EDGE_BLK, TS_W), lambda i: (i, 0)),
                pl.BlockSpec((EDGE_BLK, HID), lambda i: (i, 0)),
                pl.BlockSpec((EDGE_BLK, EDGE_DIM), lambda i: (i, 0)),
                pl.BlockSpec((EDGE_BLK, 1), lambda i: (i, 0))]
    in_specs += [_full_spec(a) for a in args[5:]]
    return pl.pallas_call(
        _edge1_body,
        grid=(E // EDGE_BLK,),
        in_specs=in_specs,
        out_specs=[pl.BlockSpec((EDGE_BLK, HID), lambda i: (i, 0)),
                   pl.BlockSpec((EDGE_BLK, HID), lambda i: (i, 0))],
        out_shape=[jax.ShapeDtypeStruct((E, HID), jnp.float32),
                   jax.ShapeDtypeStruct((E, HID), jnp.float32)],
        interpret=_INTERPRET,
    )(*args)


def _edge2_body(gd_ref, gs_ref, xw_ref, ea_ref, ew_ref, *rest):
    (k_w1a, k_w1r, k_g, k_be, k_w2, k_b2,
     v_w1a, v_w1r, v_g, v_be, v_w2, v_b2, s2_ref) = rest
    gd = gd_ref[...]
    gs = gs_ref[...]
    ea = ea_ref[...]
    rf, rel = _edge_feats(xw_ref[...], ea)
    kk = _edge_mlp(ea, rf, gd, gs, 0, k_w1a[...], k_w1r[...], k_g[...],
                   k_be[...], k_w2[...], k_b2[...])
    q = gd[:, 256:384]
    logits = ((q * kk) @ _head_sum_matrix()) * RSQRT_DH
    ex = jnp.exp(logits)
    v2 = _edge_mlp(ea, rf, gd, gs, HID, v_w1a[...], v_w1r[...], v_g[...],
                   v_be[...], v_w2[...], v_b2[...])
    vv = ex * (ew_ref[...] @ jnp.ones((1, HEADS), jnp.float32)) * v2
    vv3 = vv @ _sel(HEADS, 48, lambda r, c: c % HEADS == r)
    rel3 = rel @ _sel(16, 48, lambda r, c: c // HEADS == r)
    zpad = jnp.zeros((vv3.shape[0], 64), jnp.float32)
    s2_ref[...] = jnp.concatenate([vv3 * rel3, ex, zpad], axis=1)


def _edge2(gd, gs, xw, ea, ew, wk, wv):
    args = [gd, gs, xw, ea, ew, *wk, *wv]
    in_specs = [pl.BlockSpec((EDGE_BLK, TD_W), lambda i: (i, 0)),
                pl.BlockSpec((EDGE_BLK, TS_W), lambda i: (i, 0)),
                pl.BlockSpec((EDGE_BLK, HID), lambda i: (i, 0)),
                pl.BlockSpec((EDGE_BLK, EDGE_DIM), lambda i: (i, 0)),
                pl.BlockSpec((EDGE_BLK, 1), lambda i: (i, 0))]
    in_specs += [_full_spec(a) for a in args[5:]]
    return pl.pallas_call(
        _edge2_body,
        grid=(E // EDGE_BLK,),
        in_specs=in_specs,
        out_specs=pl.BlockSpec((EDGE_BLK, HID), lambda i: (i, 0)),
        out_shape=jax.ShapeDtypeStruct((E, HID), jnp.float32),
        interpret=_INTERPRET,
    )(*args)


def _node2_body(accw_ref, accd_ref, h_ref, inv_ref, *rest):
    (n_w1, n_b1, n_g, n_be, n_w2, n_b2,
     wk0, wk1, wk2, wk3, wv0, wv1, wv2, wv3,
     q0, q1, q2, q3, q4, q5, ho_ref, td_ref, ts_ref) = rest
    accw = accw_ref[...]
    h = h_ref[...]
    num = accw[0] + accw[1]
    accd = accd_ref[...]
    dent = (accd[0] + accd[1]) @ _sel(HEADS, HID, lambda r, c: c % HEADS == r)
    out_t = num / (dent + 1e-16)
    u = jnp.concatenate([out_t, h], axis=1)
    hdn = u @ n_w1[...] + n_b1[...]
    ho = jnp.maximum(_ln_mxu(hdn, n_g[...], n_be[...]), 0.0) @ n_w2[...] \
        + n_b2[...] + h
    ho_ref[...] = ho
    td, ts = _tables_math(
        ho, inv_ref[...],
        (wk0[...], wk1[...], wk2[...], wk3[...]),
        (wv0[...], wv1[...], wv2[...], wv3[...]),
        (q0[...], q1[...], q2[...], q3[...], q4[...], q5[...]))
    td_ref[...] = td
    ts_ref[...] = ts


def _node2(accw, accd, h, inv, wn, wk, wv, wq):
    args = [accw, accd, h, inv, *wn, *wk, *wv, *wq]
    in_specs = [pl.BlockSpec((2, NODE_BLK, HID), lambda i: (0, i, 0)),
                pl.BlockSpec((2, NODE_BLK, HEADS), lambda i: (0, i, 0)),
                pl.BlockSpec((NODE_BLK, HID), lambda i: (i, 0)),
                pl.BlockSpec((NODE_BLK, SHAPE_DIM), lambda i: (i, 0))]
    in_specs += [_full_spec(a) for a in args[4:]]
    return pl.pallas_call(
        _node2_body,
        grid=(N // NODE_BLK,),
        in_specs=in_specs,
        out_specs=[pl.BlockSpec((NODE_BLK, HID), lambda i: (i, 0)),
                   pl.BlockSpec((NODE_BLK, TD_W), lambda i: (i, 0)),
                   pl.BlockSpec((NODE_BLK, TS_W), lambda i: (i, 0))],
        out_shape=[jax.ShapeDtypeStruct((N, HID), jnp.float32),
                   jax.ShapeDtypeStruct((N, TD_W), jnp.float32),
                   jax.ShapeDtypeStruct((N, TS_W), jnp.float32)],
        interpret=_INTERPRET,
    )(*args)


def _tail_body(acc_ref, x_ref, se0_ref, se1_ref, se2_ref, wft_ref, wdt_ref,
               xo_ref):
    acc = acc_ref[...]
    x = x_ref[...]
    se = (se0_ref[...], se1_ref[...], se2_ref[...])
    den = acc[0, :, 48:64] + acc[1, :, 48:64]
    wft = wft_ref[...]
    wdt = wdt_ref[...]
    outs, Ps, Ds = [], [], []
    for c in range(3):
        num = acc[0, :, c * 16:(c + 1) * 16] + acc[1, :, c * 16:(c + 1) * 16]
        oc = num / (den + 1e-16)
        outs.append(oc)
        tmp = jnp.concatenate([x[:, c:c + 1], oc, se[c]], axis=1)
        Ps.append(tmp @ wft)
        Ds.append(tmp @ wdt)
    dot = Ps[0] * Ds[0] + Ps[1] * Ds[1] + Ps[2] * Ds[2]
    dsq = Ds[0] * Ds[0] + Ds[1] * Ds[1] + Ds[2] * Ds[2]
    coef = dot / (dsq + 1e-6)
    mask = dot >= 0.0
    deltas = []
    for c in range(3):
        neg = jnp.where(mask, Ps[c], Ps[c] - coef * Ds[c])
        res = 0.2 * Ps[c] + 0.8 * neg
        delta = jnp.mean(outs[c], axis=-1, keepdims=True) \
            + jnp.mean(res, axis=-1, keepdims=True)
        deltas.append(x[:, c:c + 1] + delta)
    xo_ref[...] = jnp.concatenate(deltas, axis=1)


def _tail(acc2, x, se0, se1, se2, wft, wdt):
    args = [acc2, x, se0, se1, se2, wft, wdt]
    in_specs = [pl.BlockSpec((2, NODE_BLK, HID), lambda i: (0, i, 0)),
                pl.BlockSpec((NODE_BLK, 3), lambda i: (i, 0)),
                pl.BlockSpec((NODE_BLK, 16), lambda i: (i, 0)),
                pl.BlockSpec((NODE_BLK, 16), lambda i: (i, 0)),
                pl.BlockSpec((NODE_BLK, 16), lambda i: (i, 0)),
                _full_spec(wft), _full_spec(wdt)]
    return pl.pallas_call(
        _tail_body,
        grid=(N // NODE_BLK,),
        in_specs=in_specs,
        out_specs=pl.BlockSpec((NODE_BLK, 3), lambda i: (i, 0)),
        out_shape=jax.ShapeDtypeStruct((N, 3), jnp.float32),
        interpret=_INTERPRET,
    )(*args)


CHUNK = 128
NROWS = E // CHUNK
NWORK = 32
ROWS_PER_W = -(-NROWS // NWORK)
NODES_PER_SUB = N // 16

_SC_MESH = plsc.VectorSubcoreMesh(core_axis_name="c", subcore_axis_name="s")


CHUNK2 = 64
NCHTOT = E // CHUNK2
NCH = -(-NCHTOT // NWORK) if NCHTOT % NWORK else NCHTOT // NWORK
NCH = 80


def _gather(td, ts, dst_m64, src_m64):
    @functools.partial(
        pl.kernel,
        out_type=[jax.ShapeDtypeStruct((E, TD_W), jnp.float32),
                  jax.ShapeDtypeStruct((E, TS_W), jnp.float32)],
        mesh=_SC_MESH,
        scratch_types=[pltpu.VMEM((NCH, CHUNK2), jnp.int32),
                       pltpu.VMEM((NCH, CHUNK2), jnp.int32),
                       pltpu.VMEM((2, CHUNK2, TD_W), jnp.float32),
                       pltpu.VMEM((2, CHUNK2, TS_W), jnp.float32),
                       pltpu.SemaphoreType.DMA,
                       pltpu.SemaphoreType.DMA,
                       pltpu.SemaphoreType.DMA,
                       pltpu.SemaphoreType.DMA],
    )
    def gk(td_hbm, ts_hbm, dm_hbm, sm_hbm, gd_hbm, gs_hbm,
           idx_d, idx_s, rows_d, rows_s, sd0, ss0, sd1, ss1):
        wid = lax.axis_index("s") * 2 + lax.axis_index("c")
        start = wid * NCH
        pltpu.sync_copy(dm_hbm.at[pl.ds(start, NCH)], idx_d)
        pltpu.sync_copy(sm_hbm.at[pl.ds(start, NCH)], idx_s)
        semd = (sd0, sd1)
        sems = (ss0, ss1)

        def issue(s, j):
            @pl.when((j < NCH) & (start + j < NCHTOT))
            def _():
                pltpu.async_copy(td_hbm.at[idx_d.at[j]], rows_d.at[s],
                                 semd[s])
                pltpu.async_copy(ts_hbm.at[idx_s.at[j]], rows_s.at[s],
                                 sems[s])

        def flush(s, j):
            @pl.when((j < NCH) & (start + j < NCHTOT))
            def _():
                pltpu.make_async_copy(td_hbm.at[idx_d.at[j]], rows_d.at[s],
                                      semd[s]).wait()
                pltpu.make_async_copy(ts_hbm.at[idx_s.at[j]], rows_s.at[s],
                                      sems[s]).wait()
                r = start + j
                pltpu.sync_copy(rows_d.at[s],
                                gd_hbm.at[pl.ds(r * CHUNK2, CHUNK2)])
                pltpu.sync_copy(rows_s.at[s],
                                gs_hbm.at[pl.ds(r * CHUNK2, CHUNK2)])

        issue(0, 0)

        @pl.loop(0, NCH // 2)
        def _(p):
            issue(1, 2 * p + 1)
            flush(0, 2 * p)
            issue(0, 2 * p + 2)
            flush(1, 2 * p + 1)

    return gk(td, ts, dst_m64, src_m64)


PCHUNK = CHUNK // 8


def _gather_rel(xpad, dst_m, src_m):
    @functools.partial(
        pl.kernel,
        out_type=jax.ShapeDtypeStruct((E, HID), jnp.float32),
        mesh=_SC_MESH,
        scratch_types=[pltpu.VMEM((ROWS_PER_W, CHUNK), jnp.int32),
                       pltpu.VMEM((ROWS_PER_W, CHUNK), jnp.int32),
                       pltpu.VMEM((2, CHUNK, 16), jnp.float32),
                       pltpu.VMEM((2, CHUNK, 16), jnp.float32),
                       pltpu.VMEM((2, CHUNK, HID), jnp.float32),
                       pltpu.SemaphoreType.DMA,
                       pltpu.SemaphoreType.DMA,
                       pltpu.SemaphoreType.DMA,
                       pltpu.SemaphoreType.DMA],
        compiler_params=pltpu.CompilerParams(use_tc_tiling_on_sc=False),
    )
    def gxk(x_hbm, dm_hbm, sm_hbm, xw_hbm,
            idx_d, idx_s, rows_d, rows_s, big, sd0, ss0, sd1, ss1):
        wid = lax.axis_index("s") * 2 + lax.axis_index("c")
        start = wid * ROWS_PER_W
        pltpu.sync_copy(dm_hbm.at[pl.ds(start, ROWS_PER_W)], idx_d)
        pltpu.sync_copy(sm_hbm.at[pl.ds(start, ROWS_PER_W)], idx_s)
        semd = (sd0, sd1)
        sems = (ss0, ss1)

        def issue(s, j):
            @pl.when((j < ROWS_PER_W) & (start + j < NROWS))
            def _():
                pltpu.async_copy(x_hbm.at[idx_d.at[j]], rows_d.at[s],
                                 semd[s])
                pltpu.async_copy(x_hbm.at[idx_s.at[j]], rows_s.at[s],
                                 sems[s])

        def flush(s, j):
            @pl.when((j < ROWS_PER_W) & (start + j < NROWS))
            def _():
                pltpu.make_async_copy(x_hbm.at[idx_d.at[j]], rows_d.at[s],
                                      semd[s]).wait()
                pltpu.make_async_copy(x_hbm.at[idx_s.at[j]], rows_s.at[s],
                                      sems[s]).wait()
                r = start + j
                pltpu.sync_copy(rows_d.at[s],
                                xw_hbm.at[pl.ds(r * CHUNK, CHUNK),
                                          pl.ds(0, 16)])
                pltpu.sync_copy(rows_s.at[s],
                                xw_hbm.at[pl.ds(r * CHUNK, CHUNK),
                                          pl.ds(16, 16)])

        issue(0, 0)

        @pl.loop(0, ROWS_PER_W // 2)
        def _(p):
            issue(1, 2 * p + 1)
            flush(0, 2 * p)
            issue(0, 2 * p + 2)
            flush(1, 2 * p + 1)

    return gxk(xpad, dst_m, src_m)


NSUB = N // 16


def _scatter_den(s1e, dst_m):
    zeros16 = jnp.zeros((NSUB, HEADS), jnp.float32)

    @functools.partial(
        pl.kernel,
        out_type=jax.ShapeDtypeStruct((2, N, HEADS), jnp.float32),
        mesh=_SC_MESH,
        scratch_types=[pltpu.VMEM((ROWS_PER_W, CHUNK), jnp.int32),
                       pltpu.VMEM((2, CHUNK, HEADS), jnp.float32),
                       pltpu.VMEM_SHARED((N, HEADS), jnp.float32),
                       pltpu.SemaphoreType.DMA,
                       pltpu.SemaphoreType.DMA],
        compiler_params=pltpu.CompilerParams(use_tc_tiling_on_sc=False),
    )
    def sk(e_hbm, dm_hbm, z_hbm, accd_hbm, idxb, bufe, shd, se0, se1):
        cid = lax.axis_index("c")
        sid = lax.axis_index("s")
        wid = sid * 2 + cid
        start = wid * ROWS_PER_W
        seme = (se0, se1)
        pltpu.sync_copy(dm_hbm.at[pl.ds(start, ROWS_PER_W)], idxb)
        pltpu.sync_copy(z_hbm, shd.at[pl.ds(sid * NSUB, NSUB)])
        plsc.subcore_barrier()

        def load(s, j):
            @pl.when((j < ROWS_PER_W) & (start + j < NROWS))
            def _():
                r = start + j
                pltpu.async_copy(e_hbm.at[pl.ds(r * CHUNK, CHUNK),
                                          pl.ds(0, HEADS)],
                                 bufe.at[s], seme[s])

        def scat(s, j):
            @pl.when((j < ROWS_PER_W) & (start + j < NROWS))
            def _():
                r = start + j
                pltpu.make_async_copy(e_hbm.at[pl.ds(r * CHUNK, CHUNK),
                                               pl.ds(0, HEADS)],
                                      bufe.at[s], seme[s]).wait()
                pltpu.sync_copy(bufe.at[s], shd.at[idxb.at[j]], add=True)

        load(0, 0)

        @pl.loop(0, ROWS_PER_W // 2)
        def _(p):
            load(1, 2 * p + 1)
            scat(0, 2 * p)
            load(0, 2 * p + 2)
            scat(1, 2 * p + 1)

        plsc.subcore_barrier()
        pltpu.sync_copy(shd.at[pl.ds(sid * NSUB, NSUB)],
                        accd_hbm.at[cid, pl.ds(sid * NSUB, NSUB)])

    return sk(s1e, dst_m, zeros16)


def _scatter2(s2, dst_m):
    zeros = jnp.zeros((NSUB, HID), jnp.float32)

    @functools.partial(
        pl.kernel,
        out_type=jax.ShapeDtypeStruct((2, N, HID), jnp.float32),
        mesh=_SC_MESH,
        scratch_types=[pltpu.VMEM((ROWS_PER_W, CHUNK), jnp.int32),
                       pltpu.VMEM((2, CHUNK, HID), jnp.float32),
                       pltpu.VMEM_SHARED((N, HID), jnp.float32),
                       pltpu.SemaphoreType.DMA,
                       pltpu.SemaphoreType.DMA],
        compiler_params=pltpu.CompilerParams(use_tc_tiling_on_sc=False),
    )
    def sk(rows_hbm, dm_hbm, z_hbm, acc_hbm, idxb, bufw, shw, sw0, sw1):
        cid = lax.axis_index("c")
        sid = lax.axis_index("s")
        wid = sid * 2 + cid
        start = wid * ROWS_PER_W
        semw = (sw0, sw1)
        pltpu.sync_copy(dm_hbm.at[pl.ds(start, ROWS_PER_W)], idxb)
        pltpu.sync_copy(z_hbm, shw.at[pl.ds(sid * NSUB, NSUB)])
        plsc.subcore_barrier()

        def load(s, j):
            @pl.when((j < ROWS_PER_W) & (start + j < NROWS))
            def _():
                r = start + j
                pltpu.async_copy(rows_hbm.at[pl.ds(r * CHUNK, CHUNK)],
                                 bufw.at[s], semw[s])

        def scat(s, j):
            @pl.when((j < ROWS_PER_W) & (start + j < NROWS))
            def _():
                r = start + j
                pltpu.make_async_copy(rows_hbm.at[pl.ds(r * CHUNK, CHUNK)],
                                      bufw.at[s], semw[s]).wait()
                pltpu.sync_copy(bufw.at[s], shw.at[idxb.at[j]], add=True)

        load(0, 0)

        @pl.loop(0, ROWS_PER_W // 2)
        def _(p):
            load(1, 2 * p + 1)
            scat(0, 2 * p)
            load(0, 2 * p + 2)
            scat(1, 2 * p + 1)

        plsc.subcore_barrier()
        pltpu.sync_copy(shw.at[pl.ds(sid * NSUB, NSUB)],
                        acc_hbm.at[cid, pl.ds(sid * NSUB, NSUB)])

    return sk(s2, dst_m, zeros)


def _prep_kv_mlp(p):
    w1 = p["W1"]
    return {
        "ea": w1[0:EDGE_DIM],
        "rf": w1[EDGE_DIM:EDGE_DIM + NG * EDGE_DIM],
        "hd": w1[84:84 + HID],
        "hs": w1[84 + HID:84 + 2 * HID],
        "iv": w1[84 + 2 * HID:],
        "b1": p["b1"].reshape(1, -1),
        "g": p["g"].reshape(1, -1),
        "be": p["be"].reshape(1, -1),
        "W2": p["W2"],
        "b2": p["b2"].reshape(1, -1),
    }


def _prep_q_mlp(p):
    return (p["W1"], p["b1"].reshape(1, -1), p["g"].reshape(1, -1),
            p["be"].reshape(1, -1), p["W2"], p["b2"].reshape(1, -1))


def kernel(h, x, edge_attr, edge_index, invar_ligand_shape, ligand_shape_emb,
           topo_out, e_w, params):
    del topo_out
    src = edge_index[0]
    dst = edge_index[1]
    dst_m = jnp.pad(dst.reshape(NROWS, CHUNK), ((0, NWORK * ROWS_PER_W - NROWS), (0, 0)))
    src_m = jnp.pad(src.reshape(NROWS, CHUNK), ((0, NWORK * ROWS_PER_W - NROWS), (0, 0)))
    dst_m64 = jnp.pad(dst.reshape(NCHTOT, CHUNK2), ((0, NWORK * NCH - NCHTOT), (0, 0)))
    src_m64 = jnp.pad(src.reshape(NCHTOT, CHUNK2), ((0, NWORK * NCH - NCHTOT), (0, 0)))
    ew = e_w.reshape(E, 1)
    xpad = jnp.pad(x, ((0, 0), (0, 13)))

    perm = np.array([(j % HEADS) * DH + j // HEADS for j in range(HID)],
                    dtype=np.int32)

    px = params["x2h"]
    hk = _prep_kv_mlp(px["hk"])
    hv = _prep_kv_mlp(px["hv"])
    hq = _prep_q_mlp(px["hq"])
    no = px["node_out"]
    n_w1 = jnp.concatenate([no["W1"][0:HID][perm], no["W1"][HID:]], axis=0)
    wn = (n_w1, no["b1"].reshape(1, -1), no["g"].reshape(1, -1),
          no["be"].reshape(1, -1), no["W2"], no["b2"].reshape(1, -1))

    ph = params["h2x"]
    xk = _prep_kv_mlp(ph["xk"])
    xv = _prep_kv_mlp(ph["xv"])
    xq = _prep_q_mlp(ph["xq"])
    wft = ph["Wf"].T
    wdt = ph["Wd"].T

    def kv_pack(m):
        return (m["hd"], m["iv"], m["b1"], m["hs"])

    xw = _gather_rel(xpad, dst_m, src_m)

    td1, ts1 = _node_tables(h, invar_ligand_shape,
                            kv_pack(hk), kv_pack(hv), hq)
    gd1, gs1 = _gather(td1, ts1, dst_m64, src_m64)
    hv_w2p = hv["W2"][:, perm]
    hv_b2p = hv["b2"][:, perm]
    s1w, s1d = _edge1(gd1, gs1, xw, edge_attr, ew,
                      (hk["ea"], hk["rf"], hk["g"], hk["be"],
                       hk["W2"], hk["b2"]),
                      (hv["ea"], hv["rf"], hv["g"], hv["be"],
                       hv_w2p, hv_b2p))
    accw1 = _scatter2(s1w, dst_m)
    accd1 = _scatter_den(s1d, dst_m)

    h_out, td2, ts2 = _node2(accw1, accd1, h, invar_ligand_shape, wn,
                             kv_pack(xk), kv_pack(xv), xq)

    gd2, gs2 = _gather(td2, ts2, dst_m64, src_m64)
    s2 = _edge2(gd2, gs2, xw, edge_attr, ew,
                (xk["ea"], xk["rf"], xk["g"], xk["be"], xk["W2"], xk["b2"]),
                (xv["ea"], xv["rf"], xv["g"], xv["be"], xv["W2"],
                 xv["b2"]))
    acc2 = _scatter2(s2, dst_m)

    se0 = ligand_shape_emb[:, :, 0]
    se1 = ligand_shape_emb[:, :, 1]
    se2 = ligand_shape_emb[:, :, 2]
    x_out = _tail(acc2, x, se0, se1, se2, wft, wdt)
    return h_out, x_out


# speedup vs baseline: 1.0987x; 1.0103x over previous
import functools

import numpy as np
import jax
import jax.numpy as jnp
from jax import lax
from jax.experimental import pallas as pl
from jax.experimental.pallas import tpu as pltpu
from jax.experimental.pallas import tpu_sc as plsc

N = 10000
E = 160000
HID = 128
HEADS = 16
DH = HID // HEADS
SHAPE_DIM = 16
EDGE_DIM = 4
NG = 20
R_MIN, R_MAX = 0.0, 10.0
RSQRT_DH = float(1.0 / np.sqrt(DH))
STEP = (R_MAX - R_MIN) / (NG - 1)
COEFF = -0.5 / STEP**2

NODE_BLK = 1000
EDGE_BLK = 3200

_INTERPRET = False

TD_W = 384
TS_W = 256


def _ln_mxu(hdn, g, be):
    o1 = jnp.full((HID, HEADS), 1.0 / HID, jnp.float32)
    o2 = jnp.full((HEADS, HID), 1.0 / HEADS, jnp.float32)
    mu = (hdn @ o1) @ o2
    ex2 = ((hdn * hdn) @ o1) @ o2
    var = ex2 - mu * mu
    return (hdn - mu) * jax.lax.rsqrt(var + 1e-5) * g + be


def _sel(rows, cols, fn):
    r = jax.lax.broadcasted_iota(jnp.int32, (rows, cols), 0)
    c = jax.lax.broadcasted_iota(jnp.int32, (rows, cols), 1)
    return fn(r, c).astype(jnp.float32)


def _full_spec(a):
    nd = a.ndim
    return pl.BlockSpec(a.shape, lambda i, *, _nd=nd: (0,) * _nd)


def _head_sum_matrix():
    j = jax.lax.broadcasted_iota(jnp.int32, (HID, HEADS), 0)
    h = jax.lax.broadcasted_iota(jnp.int32, (HID, HEADS), 1)
    return (j // DH == h).astype(jnp.float32)


def _tables_math(h, inv, wk, wv, wq):
    A_k = h @ wk[0] + inv @ wk[1] + wk[2]
    A_v = h @ wv[0] + inv @ wv[1] + wv[2]
    B_k = h @ wk[3]
    B_v = h @ wv[3]
    hdnq = h @ wq[0] + wq[1]
    q = jnp.maximum(_ln_mxu(hdnq, wq[2], wq[3]), 0.0) @ wq[4] + wq[5]
    td = jnp.concatenate([A_k, A_v, q], axis=1)
    ts = jnp.concatenate([B_k, B_v], axis=1)
    return td, ts


def _node1_body(h_ref, inv_ref, *rest):
    (wk0, wk1, wk2, wk3, wv0, wv1, wv2, wv3,
     q0, q1, q2, q3, q4, q5, td_ref, ts_ref) = rest
    td, ts = _tables_math(
        h_ref[...], inv_ref[...],
        (wk0[...], wk1[...], wk2[...], wk3[...]),
        (wv0[...], wv1[...], wv2[...], wv3[...]),
        (q0[...], q1[...], q2[...], q3[...], q4[...], q5[...]))
    td_ref[...] = td
    ts_ref[...] = ts


def _node_tables(h, inv, wk, wv, wq):
    args = [h, inv, *wk, *wv, *wq]
    in_specs = [pl.BlockSpec((NODE_BLK, h.shape[1]), lambda i: (i, 0)),
                pl.BlockSpec((NODE_BLK, SHAPE_DIM), lambda i: (i, 0))]
    in_specs += [_full_spec(a) for a in args[2:]]
    return pl.pallas_call(
        _node1_body,
        grid=(N // NODE_BLK,),
        in_specs=in_specs,
        out_specs=[pl.BlockSpec((NODE_BLK, TD_W), lambda i: (i, 0)),
                   pl.BlockSpec((NODE_BLK, TS_W), lambda i: (i, 0))],
        out_shape=[jax.ShapeDtypeStruct((N, TD_W), jnp.float32),
                   jax.ShapeDtypeStruct((N, TS_W), jnp.float32)],
        interpret=_INTERPRET,
    )(*args)


def _edge_feats(xw, ea):
    rel = xw[:, 0:16] - xw[:, 16:32]
    d2 = (rel * rel) @ jnp.ones((16, NG), jnp.float32)
    dist = jnp.sqrt(d2 + 1e-12)
    offs = jax.lax.broadcasted_iota(jnp.int32, (1, NG), 1).astype(
        jnp.float32) * STEP
    df = jnp.exp(COEFF * (dist - offs) ** 2)
    ea_b = ea @ _sel(EDGE_DIM, NG * EDGE_DIM, lambda r, c: c // NG == r)
    df_b = df @ _sel(NG, NG * EDGE_DIM, lambda r, c: c % NG == r)
    return ea_b * df_b, rel


def _edge_mlp(ea, rf, gd, gs, off, w1a, w1r, g, be, w2, b2):
    hdn = ea @ w1a + rf @ w1r + gd[:, off:off + HID] + gs[:, off:off + HID]
    return jnp.maximum(_ln_mxu(hdn, g, be), 0.0) @ w2 + b2


def _edge1_body(gd_ref, gs_ref, xw_ref, ea_ref, ew_ref, *rest):
    (k_w1a, k_w1r, k_g, k_be, k_w2, k_b2,
     v_w1a, v_w1r, v_g, v_be, v_w2p, v_b2p, s1w_ref, s1d_ref) = rest
    gd = gd_ref[...]
    gs = gs_ref[...]
    ea = ea_ref[...]
    rf, _ = _edge_feats(xw_ref[...], ea)
    kk = _edge_mlp(ea, rf, gd, gs, 0, k_w1a[...], k_w1r[...], k_g[...],
                   k_be[...], k_w2[...], k_b2[...])
    q = gd[:, 256:384]
    logits = ((q * kk) @ _head_sum_matrix()) * RSQRT_DH
    ex = jnp.exp(logits)
    vt = _edge_mlp(ea, rf, gd, gs, HID, v_w1a[...], v_w1r[...], v_g[...],
                   v_be[...], v_w2p[...], v_b2p[...])
    ext = ex @ _sel(HEADS, HID, lambda r, c: c % HEADS == r)
    extw = ext * (ew_ref[...] @ jnp.ones((1, HID), jnp.float32))
    s1w_ref[...] = extw * vt
    s1d_ref[...] = jnp.concatenate(
        [ex, jnp.zeros((ex.shape[0], HID - HEADS), jnp.float32)], axis=1)


def _edge1(gd, gs, xw, ea, ew, wk, wv):
    args = [gd, gs, xw, ea, ew, *wk, *wv]
    in_specs = [pl.BlockSpec((EDGE_BLK, TD_W), lambda i: (i, 0)),
                pl.BlockSpec((EDGE_BLK, TS_W), lambda i: (i, 0)),
                pl.BlockSpec((EDGE_BLK, HID), lambda i: (i, 0)),
                pl.BlockSpec((EDGE_BLK, EDGE_DIM), lambda i: (i, 0)),
                pl.BlockSpec((EDGE_BLK, 1), lambda i: (i, 0))]
    in_specs += [_full_spec(a) for a in args[5:]]
    return pl.pallas_call(
        _edge1_body,
        grid=(E // EDGE_BLK,),
        in_specs=in_specs,
        out_specs=[pl.BlockSpec((EDGE_BLK, HID), lambda i: (i, 0)),
                   pl.BlockSpec((EDGE_BLK, HID), lambda i: (i, 0))],
        out_shape=[jax.ShapeDtypeStruct((E, HID), jnp.float32),
                   jax.ShapeDtypeStruct((E, HID), jnp.float32)],
        interpret=_INTERPRET,
    )(*args)


def _edge2_body(gd_ref, gs_ref, xw_ref, ea_ref, ew_ref, *rest):
    (k_w1a, k_w1r, k_g, k_be, k_w2, k_b2,
     v_w1a, v_w1r, v_g, v_be, v_w2, v_b2, s2_ref) = rest
    gd = gd_ref[...]
    gs = gs_ref[...]
    ea = ea_ref[...]
    rf, rel = _edge_feats(xw_ref[...], ea)
    kk = _edge_mlp(ea, rf, gd, gs, 0, k_w1a[...], k_w1r[...], k_g[...],
                   k_be[...], k_w2[...], k_b2[...])
    q = gd[:, 256:384]
    logits = ((q * kk) @ _head_sum_matrix()) * RSQRT_DH
    ex = jnp.exp(logits)
    v2 = _edge_mlp(ea, rf, gd, gs, HID, v_w1a[...], v_w1r[...], v_g[...],
                   v_be[...], v_w2[...], v_b2[...])
    vv = ex * (ew_ref[...] @ jnp.ones((1, HEADS), jnp.float32)) * v2
    vv3 = vv @ _sel(HEADS, 48, lambda r, c: c % HEADS == r)
    rel3 = rel @ _sel(16, 48, lambda r, c: c // HEADS == r)
    zpad = jnp.zeros((vv3.shape[0], 64), jnp.float32)
    s2_ref[...] = jnp.concatenate([vv3 * rel3, ex, zpad], axis=1)


def _edge2(gd, gs, xw, ea, ew, wk, wv):
    args = [gd, gs, xw, ea, ew, *wk, *wv]
    in_specs = [pl.BlockSpec((EDGE_BLK, TD_W), lambda i: (i, 0)),
                pl.BlockSpec((EDGE_BLK, TS_W), lambda i: (i, 0)),
                pl.BlockSpec((EDGE_BLK, HID), lambda i: (i, 0)),
                pl.BlockSpec((EDGE_BLK, EDGE_DIM), lambda i: (i, 0)),
                pl.BlockSpec((EDGE_BLK, 1), lambda i: (i, 0))]
    in_specs += [_full_spec(a) for a in args[5:]]
    return pl.pallas_call(
        _edge2_body,
        grid=(E // EDGE_BLK,),
        in_specs=in_specs,
        out_specs=pl.BlockSpec((EDGE_BLK, HID), lambda i: (i, 0)),
        out_shape=jax.ShapeDtypeStruct((E, HID), jnp.float32),
        interpret=_INTERPRET,
    )(*args)


def _node2_body(accw_ref, accd_ref, h_ref, inv_ref, *rest):
    (n_w1, n_b1, n_g, n_be, n_w2, n_b2,
     wk0, wk1, wk2, wk3, wv0, wv1, wv2, wv3,
     q0, q1, q2, q3, q4, q5, ho_ref, td_ref, ts_ref) = rest
    accw = accw_ref[...]
    h = h_ref[...]
    num = accw[0] + accw[1]
    accd = accd_ref[...]
    dent = (accd[0] + accd[1]) @ _sel(HEADS, HID, lambda r, c: c % HEADS == r)
    out_t = num / (dent + 1e-16)
    u = jnp.concatenate([out_t, h], axis=1)
    hdn = u @ n_w1[...] + n_b1[...]
    ho = jnp.maximum(_ln_mxu(hdn, n_g[...], n_be[...]), 0.0) @ n_w2[...] \
        + n_b2[...] + h
    ho_ref[...] = ho
    td, ts = _tables_math(
        ho, inv_ref[...],
        (wk0[...], wk1[...], wk2[...], wk3[...]),
        (wv0[...], wv1[...], wv2[...], wv3[...]),
        (q0[...], q1[...], q2[...], q3[...], q4[...], q5[...]))
    td_ref[...] = td
    ts_ref[...] = ts


def _node2(accw, accd, h, inv, wn, wk, wv, wq):
    args = [accw, accd, h, inv, *wn, *wk, *wv, *wq]
    in_specs = [pl.BlockSpec((2, NODE_BLK, HID), lambda i: (0, i, 0)),
                pl.BlockSpec((2, NODE_BLK, HEADS), lambda i: (0, i, 0)),
                pl.BlockSpec((NODE_BLK, HID), lambda i: (i, 0)),
                pl.BlockSpec((NODE_BLK, SHAPE_DIM), lambda i: (i, 0))]
    in_specs += [_full_spec(a) for a in args[4:]]
    return pl.pallas_call(
        _node2_body,
        grid=(N // NODE_BLK,),
        in_specs=in_specs,
        out_specs=[pl.BlockSpec((NODE_BLK, HID), lambda i: (i, 0)),
                   pl.BlockSpec((NODE_BLK, TD_W), lambda i: (i, 0)),
                   pl.BlockSpec((NODE_BLK, TS_W), lambda i: (i, 0))],
        out_shape=[jax.ShapeDtypeStruct((N, HID), jnp.float32),
                   jax.ShapeDtypeStruct((N, TD_W), jnp.float32),
                   jax.ShapeDtypeStruct((N, TS_W), jnp.float32)],
        interpret=_INTERPRET,
    )(*args)


def _tail_body(acc_ref, x_ref, se0_ref, se1_ref, se2_ref, wft_ref, wdt_ref,
               xo_ref):
    acc = acc_ref[...]
    x = x_ref[...]
    se = (se0_ref[...], se1_ref[...], se2_ref[...])
    den = acc[0, :, 48:64] + acc[1, :, 48:64]
    wft = wft_ref[...]
    wdt = wdt_ref[...]
    outs, Ps, Ds = [], [], []
    for c in range(3):
        num = acc[0, :, c * 16:(c + 1) * 16] + acc[1, :, c * 16:(c + 1) * 16]
        oc = num / (den + 1e-16)
        outs.append(oc)
        tmp = jnp.concatenate([x[:, c:c + 1], oc, se[c]], axis=1)
        Ps.append(tmp @ wft)
        Ds.append(tmp @ wdt)
    dot = Ps[0] * Ds[0] + Ps[1] * Ds[1] + Ps[2] * Ds[2]
    dsq = Ds[0] * Ds[0] + Ds[1] * Ds[1] + Ds[2] * Ds[2]
    coef = dot / (dsq + 1e-6)
    mask = dot >= 0.0
    deltas = []
    for c in range(3):
        neg = jnp.where(mask, Ps[c], Ps[c] - coef * Ds[c])
        res = 0.2 * Ps[c] + 0.8 * neg
        delta = jnp.mean(outs[c], axis=-1, keepdims=True) \
            + jnp.mean(res, axis=-1, keepdims=True)
        deltas.append(x[:, c:c + 1] + delta)
    xo_ref[...] = jnp.concatenate(deltas, axis=1)


def _tail(acc2, x, se0, se1, se2, wft, wdt):
    args = [acc2, x, se0, se1, se2, wft, wdt]
    in_specs = [pl.BlockSpec((2, NODE_BLK, HID), lambda i: (0, i, 0)),
                pl.BlockSpec((NODE_BLK, 3), lambda i: (i, 0)),
                pl.BlockSpec((NODE_BLK, 16), lambda i: (i, 0)),
                pl.BlockSpec((NODE_BLK, 16), lambda i: (i, 0)),
                pl.BlockSpec((NODE_BLK, 16), lambda i: (i, 0)),
                _full_spec(wft), _full_spec(wdt)]
    return pl.pallas_call(
        _tail_body,
        grid=(N // NODE_BLK,),
        in_specs=in_specs,
        out_specs=pl.BlockSpec((NODE_BLK, 3), lambda i: (i, 0)),
        out_shape=jax.ShapeDtypeStruct((N, 3), jnp.float32),
        interpret=_INTERPRET,
    )(*args)


CHUNK = 128
NROWS = E // CHUNK
NWORK = 32
ROWS_PER_W = -(-NROWS // NWORK)
NODES_PER_SUB = N // 16

_SC_MESH = plsc.VectorSubcoreMesh(core_axis_name="c", subcore_axis_name="s")


CHUNK2 = 64
NCHTOT = E // CHUNK2
NCH = -(-NCHTOT // NWORK) if NCHTOT % NWORK else NCHTOT // NWORK
NCH = 80


def _gather(td, ts, dst_m64, src_m64):
    @functools.partial(
        pl.kernel,
        out_type=[jax.ShapeDtypeStruct((E, TD_W), jnp.float32),
                  jax.ShapeDtypeStruct((E, TS_W), jnp.float32)],
        mesh=_SC_MESH,
        scratch_types=[pltpu.VMEM((NCH, CHUNK2), jnp.int32),
                       pltpu.VMEM((NCH, CHUNK2), jnp.int32),
                       pltpu.VMEM((2, CHUNK2, TD_W), jnp.float32),
                       pltpu.VMEM((2, CHUNK2, TS_W), jnp.float32),
                       pltpu.SemaphoreType.DMA,
                       pltpu.SemaphoreType.DMA,
                       pltpu.SemaphoreType.DMA,
                       pltpu.SemaphoreType.DMA],
    )
    def gk(td_hbm, ts_hbm, dm_hbm, sm_hbm, gd_hbm, gs_hbm,
           idx_d, idx_s, rows_d, rows_s, sd0, ss0, sd1, ss1):
        wid = lax.axis_index("s") * 2 + lax.axis_index("c")
        start = wid * NCH
        pltpu.sync_copy(dm_hbm.at[pl.ds(start, NCH)], idx_d)
        pltpu.sync_copy(sm_hbm.at[pl.ds(start, NCH)], idx_s)
        semd = (sd0, sd1)
        sems = (ss0, ss1)

        def issue(s, j):
            @pl.when((j < NCH) & (start + j < NCHTOT))
            def _():
                pltpu.async_copy(td_hbm.at[idx_d.at[j]], rows_d.at[s],
                                 semd[s])
                pltpu.async_copy(ts_hbm.at[idx_s.at[j]], rows_s.at[s],
                                 sems[s])

        def flush(s, j):
            @pl.when((j < NCH) & (start + j < NCHTOT))
            def _():
                pltpu.make_async_copy(td_hbm.at[idx_d.at[j]], rows_d.at[s],
                                      semd[s]).wait()
                pltpu.make_async_copy(ts_hbm.at[idx_s.at[j]], rows_s.at[s],
                                      sems[s]).wait()
                r = start + j
                pltpu.sync_copy(rows_d.at[s],
                                gd_hbm.at[pl.ds(r * CHUNK2, CHUNK2)])
                pltpu.sync_copy(rows_s.at[s],
                                gs_hbm.at[pl.ds(r * CHUNK2, CHUNK2)])

        issue(0, 0)

        @pl.loop(0, NCH // 2)
        def _(p):
            issue(1, 2 * p + 1)
            flush(0, 2 * p)
            issue(0, 2 * p + 2)
            flush(1, 2 * p + 1)

    return gk(td, ts, dst_m64, src_m64)


PCHUNK = CHUNK // 8


def _gather_rel(xpad, dst_m, src_m):
    @functools.partial(
        pl.kernel,
        out_type=jax.ShapeDtypeStruct((E, HID), jnp.float32),
        mesh=_SC_MESH,
        scratch_types=[pltpu.VMEM((ROWS_PER_W, CHUNK), jnp.int32),
                       pltpu.VMEM((ROWS_PER_W, CHUNK), jnp.int32),
                       pltpu.VMEM((2, CHUNK, 16), jnp.float32),
                       pltpu.VMEM((2, CHUNK, 16), jnp.float32),
                       pltpu.VMEM((2, CHUNK, HID), jnp.float32),
                       pltpu.SemaphoreType.DMA,
                       pltpu.SemaphoreType.DMA,
                       pltpu.SemaphoreType.DMA,
                       pltpu.SemaphoreType.DMA],
        compiler_params=pltpu.CompilerParams(use_tc_tiling_on_sc=False),
    )
    def gxk(x_hbm, dm_hbm, sm_hbm, xw_hbm,
            idx_d, idx_s, rows_d, rows_s, big, sd0, ss0, sd1, ss1):
        wid = lax.axis_index("s") * 2 + lax.axis_index("c")
        start = wid * ROWS_PER_W
        pltpu.sync_copy(dm_hbm.at[pl.ds(start, ROWS_PER_W)], idx_d)
        pltpu.sync_copy(sm_hbm.at[pl.ds(start, ROWS_PER_W)], idx_s)
        semd = (sd0, sd1)
        sems = (ss0, ss1)

        def issue(s, j):
            @pl.when((j < ROWS_PER_W) & (start + j < NROWS))
            def _():
                pltpu.async_copy(x_hbm.at[idx_d.at[j]], rows_d.at[s],
                                 semd[s])
                pltpu.async_copy(x_hbm.at[idx_s.at[j]], rows_s.at[s],
                                 sems[s])

        def flush(s, j):
            @pl.when((j < ROWS_PER_W) & (start + j < NROWS))
            def _():
                pltpu.make_async_copy(x_hbm.at[idx_d.at[j]], rows_d.at[s],
                                      semd[s]).wait()
                pltpu.make_async_copy(x_hbm.at[idx_s.at[j]], rows_s.at[s],
                                      sems[s]).wait()
                r = start + j
                pltpu.sync_copy(rows_d.at[s],
                                xw_hbm.at[pl.ds(r * CHUNK, CHUNK),
                                          pl.ds(0, 16)])
                pltpu.sync_copy(rows_s.at[s],
                                xw_hbm.at[pl.ds(r * CHUNK, CHUNK),
                                          pl.ds(16, 16)])

        issue(0, 0)

        @pl.loop(0, ROWS_PER_W // 2)
        def _(p):
            issue(1, 2 * p + 1)
            flush(0, 2 * p)
            issue(0, 2 * p + 2)
            flush(1, 2 * p + 1)

    return gxk(xpad, dst_m, src_m)


NSUB = N // 16


def _scatter_den(s1e, dst_m):
    zeros16 = jnp.zeros((NSUB, HEADS), jnp.float32)

    @functools.partial(
        pl.kernel,
        out_type=jax.ShapeDtypeStruct((2, N, HEADS), jnp.float32),
        mesh=_SC_MESH,
        scratch_types=[pltpu.VMEM((ROWS_PER_W, CHUNK), jnp.int32),
                       pltpu.VMEM((2, CHUNK, HEADS), jnp.float32),
                       pltpu.VMEM_SHARED((N, HEADS), jnp.float32),
                       pltpu.SemaphoreType.DMA,
                       pltpu.SemaphoreType.DMA],
        compiler_params=pltpu.CompilerParams(use_tc_tiling_on_sc=False),
    )
    def sk(e_hbm, dm_hbm, z_hbm, accd_hbm, idxb, bufe, shd, se0, se1):
        cid = lax.axis_index("c")
        sid = lax.axis_index("s")
        wid = sid * 2 + cid
        start = wid * ROWS_PER_W
        seme = (se0, se1)
        pltpu.sync_copy(dm_hbm.at[pl.ds(start, ROWS_PER_W)], idxb)
        pltpu.sync_copy(z_hbm, shd.at[pl.ds(sid * NSUB, NSUB)])
        plsc.subcore_barrier()

        def load(s, j):
            @pl.when((j < ROWS_PER_W) & (start + j < NROWS))
            def _():
                r = start + j
                pltpu.async_copy(e_hbm.at[pl.ds(r * CHUNK, CHUNK),
                                          pl.ds(0, HEADS)],
                                 bufe.at[s], seme[s])

        def scat(s, j):
            @pl.when((j < ROWS_PER_W) & (start + j < NROWS))
            def _():
                r = start + j
                pltpu.make_async_copy(e_hbm.at[pl.ds(r * CHUNK, CHUNK),
                                               pl.ds(0, HEADS)],
                                      bufe.at[s], seme[s]).wait()
                pltpu.sync_copy(bufe.at[s], shd.at[idxb.at[j]], add=True)

        load(0, 0)

        @pl.loop(0, ROWS_PER_W // 2)
        def _(p):
            load(1, 2 * p + 1)
            scat(0, 2 * p)
            load(0, 2 * p + 2)
            scat(1, 2 * p + 1)

        plsc.subcore_barrier()
        pltpu.sync_copy(shd.at[pl.ds(sid * NSUB, NSUB)],
                        accd_hbm.at[cid, pl.ds(sid * NSUB, NSUB)])

    return sk(s1e, dst_m, zeros16)


def _scatter2(s2, dst_m):
    zeros = jnp.zeros((NSUB, HID), jnp.float32)

    @functools.partial(
        pl.kernel,
        out_type=jax.ShapeDtypeStruct((2, N, HID), jnp.float32),
        mesh=_SC_MESH,
        scratch_types=[pltpu.VMEM((ROWS_PER_W, CHUNK), jnp.int32),
                       pltpu.VMEM((2, CHUNK, HID), jnp.float32),
                       pltpu.VMEM_SHARED((N, HID), jnp.float32),
                       pltpu.SemaphoreType.DMA,
                       pltpu.SemaphoreType.DMA],
        compiler_params=pltpu.CompilerParams(use_tc_tiling_on_sc=False),
    )
    def sk(rows_hbm, dm_hbm, z_hbm, acc_hbm, idxb, bufw, shw, sw0, sw1):
        cid = lax.axis_index("c")
        sid = lax.axis_index("s")
        wid = sid * 2 + cid
        start = wid * ROWS_PER_W
        semw = (sw0, sw1)
        pltpu.sync_copy(dm_hbm.at[pl.ds(start, ROWS_PER_W)], idxb)
        pltpu.sync_copy(z_hbm, shw.at[pl.ds(sid * NSUB, NSUB)])
        plsc.subcore_barrier()

        def load(s, j):
            @pl.when((j < ROWS_PER_W) & (start + j < NROWS))
            def _():
                r = start + j
                pltpu.async_copy(rows_hbm.at[pl.ds(r * CHUNK, CHUNK)],
                                 bufw.at[s], semw[s])

        def scat(s, j):
            @pl.when((j < ROWS_PER_W) & (start + j < NROWS))
            def _():
                r = start + j
                pltpu.make_async_copy(rows_hbm.at[pl.ds(r * CHUNK, CHUNK)],
                                      bufw.at[s], semw[s]).wait()
                pltpu.sync_copy(bufw.at[s], shw.at[idxb.at[j]], add=True)

        load(0, 0)

        @pl.loop(0, ROWS_PER_W // 2)
        def _(p):
            load(1, 2 * p + 1)
            scat(0, 2 * p)
            load(0, 2 * p + 2)
            scat(1, 2 * p + 1)

        plsc.subcore_barrier()
        pltpu.sync_copy(shw.at[pl.ds(sid * NSUB, NSUB)],
                        acc_hbm.at[cid, pl.ds(sid * NSUB, NSUB)])

    return sk(s2, dst_m, zeros)


def _prep_kv_mlp(p):
    w1 = p["W1"]
    return {
        "ea": w1[0:EDGE_DIM],
        "rf": w1[EDGE_DIM:EDGE_DIM + NG * EDGE_DIM],
        "hd": w1[84:84 + HID],
        "hs": w1[84 + HID:84 + 2 * HID],
        "iv": w1[84 + 2 * HID:],
        "b1": p["b1"].reshape(1, -1),
        "g": p["g"].reshape(1, -1),
        "be": p["be"].reshape(1, -1),
        "W2": p["W2"],
        "b2": p["b2"].reshape(1, -1),
    }


def _prep_q_mlp(p):
    return (p["W1"], p["b1"].reshape(1, -1), p["g"].reshape(1, -1),
            p["be"].reshape(1, -1), p["W2"], p["b2"].reshape(1, -1))


def kernel(h, x, edge_attr, edge_index, invar_ligand_shape, ligand_shape_emb,
           topo_out, e_w, params):
    del topo_out
    src = edge_index[0]
    dst = edge_index[1]
    dst_m = jnp.pad(dst.reshape(NROWS, CHUNK), ((0, NWORK * ROWS_PER_W - NROWS), (0, 0)))
    src_m = jnp.pad(src.reshape(NROWS, CHUNK), ((0, NWORK * ROWS_PER_W - NROWS), (0, 0)))
    dst_m64 = jnp.pad(dst.reshape(NCHTOT, CHUNK2), ((0, NWORK * NCH - NCHTOT), (0, 0)))
    src_m64 = jnp.pad(src.reshape(NCHTOT, CHUNK2), ((0, NWORK * NCH - NCHTOT), (0, 0)))
    ew = e_w.reshape(E, 1)
    xpad = jnp.pad(x, ((0, 0), (0, 13)))

    perm = np.array([(j % HEADS) * DH + j // HEADS for j in range(HID)],
                    dtype=np.int32)

    px = params["x2h"]
    hk = _prep_kv_mlp(px["hk"])
    hv = _prep_kv_mlp(px["hv"])
    hq = _prep_q_mlp(px["hq"])
    no = px["node_out"]
    n_w1 = jnp.concatenate([no["W1"][0:HID][perm], no["W1"][HID:]], axis=0)
    wn = (n_w1, no["b1"].reshape(1, -1), no["g"].reshape(1, -1),
          no["be"].reshape(1, -1), no["W2"], no["b2"].reshape(1, -1))

    ph = params["h2x"]
    xk = _prep_kv_mlp(ph["xk"])
    xv = _prep_kv_mlp(ph["xv"])
    xq = _prep_q_mlp(ph["xq"])
    wft = ph["Wf"].T
    wdt = ph["Wd"].T

    def kv_pack(m):
        return (m["hd"], m["iv"], m["b1"], m["hs"])

    xw = _gather_rel(xpad, dst_m, src_m)

    td1, ts1 = _node_tables(h, invar_ligand_shape,
                            kv_pack(hk), kv_pack(hv), hq)
    gd1, gs1 = _gather(td1, ts1, dst_m64, src_m64)
    hv_w2p = hv["W2"][:, perm]
    hv_b2p = hv["b2"][:, perm]
    s1w, s1d = _edge1(gd1, gs1, xw, edge_attr, ew,
                      (hk["ea"], hk["rf"], hk["g"], hk["be"],
                       hk["W2"], hk["b2"]),
                      (hv["ea"], hv["rf"], hv["g"], hv["be"],
                       hv_w2p, hv_b2p))
    accw1 = _scatter2(s1w, dst_m)
    accd1 = _scatter_den(s1d, dst_m)

    h_out, td2, ts2 = _node2(accw1, accd1, h, invar_ligand_shape, wn,
                             kv_pack(xk), kv_pack(xv), xq)

    gd2, gs2 = _gather(td2, ts2, dst_m64, src_m64)
    s2 = _edge2(gd2, gs2, xw, edge_attr, ew,
                (xk["ea"], xk["rf"], xk["g"], xk["be"], xk["W2"], xk["b2"]),
                (xv["ea"], xv["rf"], xv["g"], xv["be"], xv["W2"],
                 xv["b2"]))
    acc2 = _scatter2(s2, dst_m)

    se0 = ligand_shape_emb[:, :, 0]
    se1 = ligand_shape_emb[:, :, 1]
    se2 = ligand_shape_emb[:, :, 2]
    x_out = _tail(acc2, x, se0, se1, se2, wft, wdt)
    return h_out, x_out


# speedup vs baseline: 1.1102x; 1.0104x over previous
import functools

import numpy as np
import jax
import jax.numpy as jnp
from jax import lax
from jax.experimental import pallas as pl
from jax.experimental.pallas import tpu as pltpu
from jax.experimental.pallas import tpu_sc as plsc

N = 10000
E = 160000
HID = 128
HEADS = 16
DH = HID // HEADS
SHAPE_DIM = 16
EDGE_DIM = 4
NG = 20
R_MIN, R_MAX = 0.0, 10.0
RSQRT_DH = float(1.0 / np.sqrt(DH))
STEP = (R_MAX - R_MIN) / (NG - 1)
COEFF = -0.5 / STEP**2

NODE_BLK = 1000
EDGE_BLK = 4000

_INTERPRET = False

TD_W = 384
TS_W = 256


def _ln_mxu(hdn, g, be):
    o1 = jnp.full((HID, HEADS), 1.0 / HID, jnp.float32)
    o2 = jnp.full((HEADS, HID), 1.0 / HEADS, jnp.float32)
    mu = (hdn @ o1) @ o2
    ex2 = ((hdn * hdn) @ o1) @ o2
    var = ex2 - mu * mu
    return (hdn - mu) * jax.lax.rsqrt(var + 1e-5) * g + be


def _sel(rows, cols, fn):
    r = jax.lax.broadcasted_iota(jnp.int32, (rows, cols), 0)
    c = jax.lax.broadcasted_iota(jnp.int32, (rows, cols), 1)
    return fn(r, c).astype(jnp.float32)


def _full_spec(a):
    nd = a.ndim
    return pl.BlockSpec(a.shape, lambda i, *, _nd=nd: (0,) * _nd)


def _head_sum_matrix():
    j = jax.lax.broadcasted_iota(jnp.int32, (HID, HEADS), 0)
    h = jax.lax.broadcasted_iota(jnp.int32, (HID, HEADS), 1)
    return (j // DH == h).astype(jnp.float32)


def _tables_math(h, inv, wk, wv, wq):
    A_k = h @ wk[0] + inv @ wk[1] + wk[2]
    A_v = h @ wv[0] + inv @ wv[1] + wv[2]
    B_k = h @ wk[3]
    B_v = h @ wv[3]
    hdnq = h @ wq[0] + wq[1]
    q = jnp.maximum(_ln_mxu(hdnq, wq[2], wq[3]), 0.0) @ wq[4] + wq[5]
    td = jnp.concatenate([A_k, A_v, q], axis=1)
    ts = jnp.concatenate([B_k, B_v], axis=1)
    return td, ts


def _node1_body(h_ref, inv_ref, *rest):
    (wk0, wk1, wk2, wk3, wv0, wv1, wv2, wv3,
     q0, q1, q2, q3, q4, q5, td_ref, ts_ref) = rest
    td, ts = _tables_math(
        h_ref[...], inv_ref[...],
        (wk0[...], wk1[...], wk2[...], wk3[...]),
        (wv0[...], wv1[...], wv2[...], wv3[...]),
        (q0[...], q1[...], q2[...], q3[...], q4[...], q5[...]))
    td_ref[...] = td
    ts_ref[...] = ts


def _node_tables(h, inv, wk, wv, wq):
    args = [h, inv, *wk, *wv, *wq]
    in_specs = [pl.BlockSpec((NODE_BLK, h.shape[1]), lambda i: (i, 0)),
                pl.BlockSpec((NODE_BLK, SHAPE_DIM), lambda i: (i, 0))]
    in_specs += [_full_spec(a) for a in args[2:]]
    return pl.pallas_call(
        _node1_body,
        grid=(N // NODE_BLK,),
        in_specs=in_specs,
        out_specs=[pl.BlockSpec((NODE_BLK, TD_W), lambda i: (i, 0)),
                   pl.BlockSpec((NODE_BLK, TS_W), lambda i: (i, 0))],
        out_shape=[jax.ShapeDtypeStruct((N, TD_W), jnp.float32),
                   jax.ShapeDtypeStruct((N, TS_W), jnp.float32)],
        interpret=_INTERPRET,
    )(*args)


def _edge_feats(xw, ea):
    rel = xw[:, 0:16] - xw[:, 16:32]
    d2 = (rel * rel) @ jnp.ones((16, NG), jnp.float32)
    dist = jnp.sqrt(d2 + 1e-12)
    offs = jax.lax.broadcasted_iota(jnp.int32, (1, NG), 1).astype(
        jnp.float32) * STEP
    df = jnp.exp(COEFF * (dist - offs) ** 2)
    ea_b = ea @ _sel(EDGE_DIM, NG * EDGE_DIM, lambda r, c: c // NG == r)
    df_b = df @ _sel(NG, NG * EDGE_DIM, lambda r, c: c % NG == r)
    return ea_b * df_b, rel


def _edge_mlp(ea, rf, gd, gs, off, w1a, w1r, g, be, w2, b2):
    hdn = ea @ w1a + rf @ w1r + gd[:, off:off + HID] + gs[:, off:off + HID]
    return jnp.maximum(_ln_mxu(hdn, g, be), 0.0) @ w2 + b2


def _edge1_body(gd_ref, gs_ref, xw_ref, ea_ref, ew_ref, *rest):
    (k_w1a, k_w1r, k_g, k_be, k_w2, k_b2,
     v_w1a, v_w1r, v_g, v_be, v_w2p, v_b2p, s1w_ref, s1d_ref) = rest
    gd = gd_ref[...]
    gs = gs_ref[...]
    ea = ea_ref[...]
    rf, _ = _edge_feats(xw_ref[...], ea)
    kk = _edge_mlp(ea, rf, gd, gs, 0, k_w1a[...], k_w1r[...], k_g[...],
                   k_be[...], k_w2[...], k_b2[...])
    q = gd[:, 256:384]
    logits = ((q * kk) @ _head_sum_matrix()) * RSQRT_DH
    ex = jnp.exp(logits)
    vt = _edge_mlp(ea, rf, gd, gs, HID, v_w1a[...], v_w1r[...], v_g[...],
                   v_be[...], v_w2p[...], v_b2p[...])
    ext = ex @ _sel(HEADS, HID, lambda r, c: c % HEADS == r)
    extw = ext * (ew_ref[...] @ jnp.ones((1, HID), jnp.float32))
    s1w_ref[...] = extw * vt
    s1d_ref[...] = jnp.concatenate(
        [ex, jnp.zeros((ex.shape[0], HID - HEADS), jnp.float32)], axis=1)


def _edge1(gd, gs, xw, ea, ew, wk, wv):
    args = [gd, gs, xw, ea, ew, *wk, *wv]
    in_specs = [pl.BlockSpec((EDGE_BLK, TD_W), lambda i: (i, 0)),
                pl.BlockSpec((EDGE_BLK, TS_W), lambda i: (i, 0)),
                pl.BlockSpec((EDGE_BLK, HID), lambda i: (i, 0)),
                pl.BlockSpec((EDGE_BLK, EDGE_DIM), lambda i: (i, 0)),
                pl.BlockSpec((EDGE_BLK, 1), lambda i: (i, 0))]
    in_specs += [_full_spec(a) for a in args[5:]]
    return pl.pallas_call(
        _edge1_body,
        grid=(E // EDGE_BLK,),
        in_specs=in_specs,
        out_specs=[pl.BlockSpec((EDGE_BLK, HID), lambda i: (i, 0)),
                   pl.BlockSpec((EDGE_BLK, HID), lambda i: (i, 0))],
        out_shape=[jax.ShapeDtypeStruct((E, HID), jnp.float32),
                   jax.ShapeDtypeStruct((E, HID), jnp.float32)],
        interpret=_INTERPRET,
    )(*args)


def _edge2_body(gd_ref, gs_ref, xw_ref, ea_ref, ew_ref, *rest):
    (k_w1a, k_w1r, k_g, k_be, k_w2, k_b2,
     v_w1a, v_w1r, v_g, v_be, v_w2, v_b2, s2_ref) = rest
    gd = gd_ref[...]
    gs = gs_ref[...]
    ea = ea_ref[...]
    rf, rel = _edge_feats(xw_ref[...], ea)
    kk = _edge_mlp(ea, rf, gd, gs, 0, k_w1a[...], k_w1r[...], k_g[...],
                   k_be[...], k_w2[...], k_b2[...])
    q = gd[:, 256:384]
    logits = ((q * kk) @ _head_sum_matrix()) * RSQRT_DH
    ex = jnp.exp(logits)
    v2 = _edge_mlp(ea, rf, gd, gs, HID, v_w1a[...], v_w1r[...], v_g[...],
                   v_be[...], v_w2[...], v_b2[...])
    vv = ex * (ew_ref[...] @ jnp.ones((1, HEADS), jnp.float32)) * v2
    vv3 = vv @ _sel(HEADS, 48, lambda r, c: c % HEADS == r)
    rel3 = rel @ _sel(16, 48, lambda r, c: c // HEADS == r)
    zpad = jnp.zeros((vv3.shape[0], 64), jnp.float32)
    s2_ref[...] = jnp.concatenate([vv3 * rel3, ex, zpad], axis=1)


def _edge2(gd, gs, xw, ea, ew, wk, wv):
    args = [gd, gs, xw, ea, ew, *wk, *wv]
    in_specs = [pl.BlockSpec((EDGE_BLK, TD_W), lambda i: (i, 0)),
                pl.BlockSpec((EDGE_BLK, TS_W), lambda i: (i, 0)),
                pl.BlockSpec((EDGE_BLK, HID), lambda i: (i, 0)),
                pl.BlockSpec((EDGE_BLK, EDGE_DIM), lambda i: (i, 0)),
                pl.BlockSpec((EDGE_BLK, 1), lambda i: (i, 0))]
    in_specs += [_full_spec(a) for a in args[5:]]
    return pl.pallas_call(
        _edge2_body,
        grid=(E // EDGE_BLK,),
        in_specs=in_specs,
        out_specs=pl.BlockSpec((EDGE_BLK, HID), lambda i: (i, 0)),
        out_shape=jax.ShapeDtypeStruct((E, HID), jnp.float32),
        interpret=_INTERPRET,
    )(*args)


def _node2_body(accw_ref, accd_ref, h_ref, inv_ref, *rest):
    (n_w1, n_b1, n_g, n_be, n_w2, n_b2,
     wk0, wk1, wk2, wk3, wv0, wv1, wv2, wv3,
     q0, q1, q2, q3, q4, q5, ho_ref, td_ref, ts_ref) = rest
    accw = accw_ref[...]
    h = h_ref[...]
    num = accw[0] + accw[1]
    accd = accd_ref[...]
    dent = (accd[0] + accd[1]) @ _sel(HEADS, HID, lambda r, c: c % HEADS == r)
    out_t = num / (dent + 1e-16)
    u = jnp.concatenate([out_t, h], axis=1)
    hdn = u @ n_w1[...] + n_b1[...]
    ho = jnp.maximum(_ln_mxu(hdn, n_g[...], n_be[...]), 0.0) @ n_w2[...] \
        + n_b2[...] + h
    ho_ref[...] = ho
    td, ts = _tables_math(
        ho, inv_ref[...],
        (wk0[...], wk1[...], wk2[...], wk3[...]),
        (wv0[...], wv1[...], wv2[...], wv3[...]),
        (q0[...], q1[...], q2[...], q3[...], q4[...], q5[...]))
    td_ref[...] = td
    ts_ref[...] = ts


def _node2(accw, accd, h, inv, wn, wk, wv, wq):
    args = [accw, accd, h, inv, *wn, *wk, *wv, *wq]
    in_specs = [pl.BlockSpec((2, NODE_BLK, HID), lambda i: (0, i, 0)),
                pl.BlockSpec((2, NODE_BLK, HEADS), lambda i: (0, i, 0)),
                pl.BlockSpec((NODE_BLK, HID), lambda i: (i, 0)),
                pl.BlockSpec((NODE_BLK, SHAPE_DIM), lambda i: (i, 0))]
    in_specs += [_full_spec(a) for a in args[4:]]
    return pl.pallas_call(
        _node2_body,
        grid=(N // NODE_BLK,),
        in_specs=in_specs,
        out_specs=[pl.BlockSpec((NODE_BLK, HID), lambda i: (i, 0)),
                   pl.BlockSpec((NODE_BLK, TD_W), lambda i: (i, 0)),
                   pl.BlockSpec((NODE_BLK, TS_W), lambda i: (i, 0))],
        out_shape=[jax.ShapeDtypeStruct((N, HID), jnp.float32),
                   jax.ShapeDtypeStruct((N, TD_W), jnp.float32),
                   jax.ShapeDtypeStruct((N, TS_W), jnp.float32)],
        interpret=_INTERPRET,
    )(*args)


def _tail_body(acc_ref, x_ref, se0_ref, se1_ref, se2_ref, wft_ref, wdt_ref,
               xo_ref):
    acc = acc_ref[...]
    x = x_ref[...]
    se = (se0_ref[...], se1_ref[...], se2_ref[...])
    den = acc[0, :, 48:64] + acc[1, :, 48:64]
    wft = wft_ref[...]
    wdt = wdt_ref[...]
    outs, Ps, Ds = [], [], []
    for c in range(3):
        num = acc[0, :, c * 16:(c + 1) * 16] + acc[1, :, c * 16:(c + 1) * 16]
        oc = num / (den + 1e-16)
        outs.append(oc)
        tmp = jnp.concatenate([x[:, c:c + 1], oc, se[c]], axis=1)
        Ps.append(tmp @ wft)
        Ds.append(tmp @ wdt)
    dot = Ps[0] * Ds[0] + Ps[1] * Ds[1] + Ps[2] * Ds[2]
    dsq = Ds[0] * Ds[0] + Ds[1] * Ds[1] + Ds[2] * Ds[2]
    coef = dot / (dsq + 1e-6)
    mask = dot >= 0.0
    deltas = []
    for c in range(3):
        neg = jnp.where(mask, Ps[c], Ps[c] - coef * Ds[c])
        res = 0.2 * Ps[c] + 0.8 * neg
        delta = jnp.mean(outs[c], axis=-1, keepdims=True) \
            + jnp.mean(res, axis=-1, keepdims=True)
        deltas.append(x[:, c:c + 1] + delta)
    xo_ref[...] = jnp.concatenate(deltas, axis=1)


def _tail(acc2, x, se0, se1, se2, wft, wdt):
    args = [acc2, x, se0, se1, se2, wft, wdt]
    in_specs = [pl.BlockSpec((2, NODE_BLK, HID), lambda i: (0, i, 0)),
                pl.BlockSpec((NODE_BLK, 3), lambda i: (i, 0)),
                pl.BlockSpec((NODE_BLK, 16), lambda i: (i, 0)),
                pl.BlockSpec((NODE_BLK, 16), lambda i: (i, 0)),
                pl.BlockSpec((NODE_BLK, 16), lambda i: (i, 0)),
                _full_spec(wft), _full_spec(wdt)]
    return pl.pallas_call(
        _tail_body,
        grid=(N // NODE_BLK,),
        in_specs=in_specs,
        out_specs=pl.BlockSpec((NODE_BLK, 3), lambda i: (i, 0)),
        out_shape=jax.ShapeDtypeStruct((N, 3), jnp.float32),
        interpret=_INTERPRET,
    )(*args)


CHUNK = 128
NROWS = E // CHUNK
NWORK = 32
ROWS_PER_W = -(-NROWS // NWORK)
NODES_PER_SUB = N // 16

_SC_MESH = plsc.VectorSubcoreMesh(core_axis_name="c", subcore_axis_name="s")


CHUNK2 = 64
NCHTOT = E // CHUNK2
NCH = -(-NCHTOT // NWORK) if NCHTOT % NWORK else NCHTOT // NWORK
NCH = 80


def _gather(td, ts, dst_m64, src_m64):
    @functools.partial(
        pl.kernel,
        out_type=[jax.ShapeDtypeStruct((E, TD_W), jnp.float32),
                  jax.ShapeDtypeStruct((E, TS_W), jnp.float32)],
        mesh=_SC_MESH,
        scratch_types=[pltpu.VMEM((NCH, CHUNK2), jnp.int32),
                       pltpu.VMEM((NCH, CHUNK2), jnp.int32),
                       pltpu.VMEM((2, CHUNK2, TD_W), jnp.float32),
                       pltpu.VMEM((2, CHUNK2, TS_W), jnp.float32),
                       pltpu.SemaphoreType.DMA,
                       pltpu.SemaphoreType.DMA,
                       pltpu.SemaphoreType.DMA,
                       pltpu.SemaphoreType.DMA],
    )
    def gk(td_hbm, ts_hbm, dm_hbm, sm_hbm, gd_hbm, gs_hbm,
           idx_d, idx_s, rows_d, rows_s, sd0, ss0, sd1, ss1):
        wid = lax.axis_index("s") * 2 + lax.axis_index("c")
        start = wid * NCH
        pltpu.sync_copy(dm_hbm.at[pl.ds(start, NCH)], idx_d)
        pltpu.sync_copy(sm_hbm.at[pl.ds(start, NCH)], idx_s)
        semd = (sd0, sd1)
        sems = (ss0, ss1)

        def issue(s, j):
            @pl.when((j < NCH) & (start + j < NCHTOT))
            def _():
                pltpu.async_copy(td_hbm.at[idx_d.at[j]], rows_d.at[s],
                                 semd[s])
                pltpu.async_copy(ts_hbm.at[idx_s.at[j]], rows_s.at[s],
                                 sems[s])

        def flush(s, j):
            @pl.when((j < NCH) & (start + j < NCHTOT))
            def _():
                pltpu.make_async_copy(td_hbm.at[idx_d.at[j]], rows_d.at[s],
                                      semd[s]).wait()
                pltpu.make_async_copy(ts_hbm.at[idx_s.at[j]], rows_s.at[s],
                                      sems[s]).wait()
                r = start + j
                pltpu.sync_copy(rows_d.at[s],
                                gd_hbm.at[pl.ds(r * CHUNK2, CHUNK2)])
                pltpu.sync_copy(rows_s.at[s],
                                gs_hbm.at[pl.ds(r * CHUNK2, CHUNK2)])

        issue(0, 0)

        @pl.loop(0, NCH // 2)
        def _(p):
            issue(1, 2 * p + 1)
            flush(0, 2 * p)
            issue(0, 2 * p + 2)
            flush(1, 2 * p + 1)

    return gk(td, ts, dst_m64, src_m64)


PCHUNK = CHUNK // 8


def _gather_rel(xpad, dst_m, src_m):
    @functools.partial(
        pl.kernel,
        out_type=jax.ShapeDtypeStruct((E, HID), jnp.float32),
        mesh=_SC_MESH,
        scratch_types=[pltpu.VMEM((ROWS_PER_W, CHUNK), jnp.int32),
                       pltpu.VMEM((ROWS_PER_W, CHUNK), jnp.int32),
                       pltpu.VMEM((2, CHUNK, 16), jnp.float32),
                       pltpu.VMEM((2, CHUNK, 16), jnp.float32),
                       pltpu.VMEM((2, CHUNK, HID), jnp.float32),
                       pltpu.SemaphoreType.DMA,
                       pltpu.SemaphoreType.DMA,
                       pltpu.SemaphoreType.DMA,
                       pltpu.SemaphoreType.DMA],
        compiler_params=pltpu.CompilerParams(use_tc_tiling_on_sc=False),
    )
    def gxk(x_hbm, dm_hbm, sm_hbm, xw_hbm,
            idx_d, idx_s, rows_d, rows_s, big, sd0, ss0, sd1, ss1):
        wid = lax.axis_index("s") * 2 + lax.axis_index("c")
        start = wid * ROWS_PER_W
        pltpu.sync_copy(dm_hbm.at[pl.ds(start, ROWS_PER_W)], idx_d)
        pltpu.sync_copy(sm_hbm.at[pl.ds(start, ROWS_PER_W)], idx_s)
        semd = (sd0, sd1)
        sems = (ss0, ss1)

        def issue(s, j):
            @pl.when((j < ROWS_PER_W) & (start + j < NROWS))
            def _():
                pltpu.async_copy(x_hbm.at[idx_d.at[j]], rows_d.at[s],
                                 semd[s])
                pltpu.async_copy(x_hbm.at[idx_s.at[j]], rows_s.at[s],
                                 sems[s])

        def flush(s, j):
            @pl.when((j < ROWS_PER_W) & (start + j < NROWS))
            def _():
                pltpu.make_async_copy(x_hbm.at[idx_d.at[j]], rows_d.at[s],
                                      semd[s]).wait()
                pltpu.make_async_copy(x_hbm.at[idx_s.at[j]], rows_s.at[s],
                                      sems[s]).wait()
                r = start + j
                pltpu.sync_copy(rows_d.at[s],
                                xw_hbm.at[pl.ds(r * CHUNK, CHUNK),
                                          pl.ds(0, 16)])
                pltpu.sync_copy(rows_s.at[s],
                                xw_hbm.at[pl.ds(r * CHUNK, CHUNK),
                                          pl.ds(16, 16)])

        issue(0, 0)

        @pl.loop(0, ROWS_PER_W // 2)
        def _(p):
            issue(1, 2 * p + 1)
            flush(0, 2 * p)
            issue(0, 2 * p + 2)
            flush(1, 2 * p + 1)

    return gxk(xpad, dst_m, src_m)


NSUB = N // 16


def _scatter_den(s1e, dst_m):
    zeros16 = jnp.zeros((NSUB, HEADS), jnp.float32)

    @functools.partial(
        pl.kernel,
        out_type=jax.ShapeDtypeStruct((2, N, HEADS), jnp.float32),
        mesh=_SC_MESH,
        scratch_types=[pltpu.VMEM((ROWS_PER_W, CHUNK), jnp.int32),
                       pltpu.VMEM((2, CHUNK, HEADS), jnp.float32),
                       pltpu.VMEM_SHARED((N, HEADS), jnp.float32),
                       pltpu.SemaphoreType.DMA,
                       pltpu.SemaphoreType.DMA],
        compiler_params=pltpu.CompilerParams(use_tc_tiling_on_sc=False),
    )
    def sk(e_hbm, dm_hbm, z_hbm, accd_hbm, idxb, bufe, shd, se0, se1):
        cid = lax.axis_index("c")
        sid = lax.axis_index("s")
        wid = sid * 2 + cid
        start = wid * ROWS_PER_W
        seme = (se0, se1)
        pltpu.sync_copy(dm_hbm.at[pl.ds(start, ROWS_PER_W)], idxb)
        pltpu.sync_copy(z_hbm, shd.at[pl.ds(sid * NSUB, NSUB)])
        plsc.subcore_barrier()

        def load(s, j):
            @pl.when((j < ROWS_PER_W) & (start + j < NROWS))
            def _():
                r = start + j
                pltpu.async_copy(e_hbm.at[pl.ds(r * CHUNK, CHUNK),
                                          pl.ds(0, HEADS)],
                                 bufe.at[s], seme[s])

        def scat(s, j):
            @pl.when((j < ROWS_PER_W) & (start + j < NROWS))
            def _():
                r = start + j
                pltpu.make_async_copy(e_hbm.at[pl.ds(r * CHUNK, CHUNK),
                                               pl.ds(0, HEADS)],
                                      bufe.at[s], seme[s]).wait()
                pltpu.sync_copy(bufe.at[s], shd.at[idxb.at[j]], add=True)

        load(0, 0)

        @pl.loop(0, ROWS_PER_W // 2)
        def _(p):
            load(1, 2 * p + 1)
            scat(0, 2 * p)
            load(0, 2 * p + 2)
            scat(1, 2 * p + 1)

        plsc.subcore_barrier()
        pltpu.sync_copy(shd.at[pl.ds(sid * NSUB, NSUB)],
                        accd_hbm.at[cid, pl.ds(sid * NSUB, NSUB)])

    return sk(s1e, dst_m, zeros16)


def _scatter2(s2, dst_m):
    zeros = jnp.zeros((NSUB, HID), jnp.float32)

    @functools.partial(
        pl.kernel,
        out_type=jax.ShapeDtypeStruct((2, N, HID), jnp.float32),
        mesh=_SC_MESH,
        scratch_types=[pltpu.VMEM((ROWS_PER_W, CHUNK), jnp.int32),
                       pltpu.VMEM((2, CHUNK, HID), jnp.float32),
                       pltpu.VMEM_SHARED((N, HID), jnp.float32),
                       pltpu.SemaphoreType.DMA,
                       pltpu.SemaphoreType.DMA],
        compiler_params=pltpu.CompilerParams(use_tc_tiling_on_sc=False),
    )
    def sk(rows_hbm, dm_hbm, z_hbm, acc_hbm, idxb, bufw, shw, sw0, sw1):
        cid = lax.axis_index("c")
        sid = lax.axis_index("s")
        wid = sid * 2 + cid
        start = wid * ROWS_PER_W
        semw = (sw0, sw1)
        pltpu.sync_copy(dm_hbm.at[pl.ds(start, ROWS_PER_W)], idxb)
        pltpu.sync_copy(z_hbm, shw.at[pl.ds(sid * NSUB, NSUB)])
        plsc.subcore_barrier()

        def load(s, j):
            @pl.when((j < ROWS_PER_W) & (start + j < NROWS))
            def _():
                r = start + j
                pltpu.async_copy(rows_hbm.at[pl.ds(r * CHUNK, CHUNK)],
                                 bufw.at[s], semw[s])

        def scat(s, j):
            @pl.when((j < ROWS_PER_W) & (start + j < NROWS))
            def _():
                r = start + j
                pltpu.make_async_copy(rows_hbm.at[pl.ds(r * CHUNK, CHUNK)],
                                      bufw.at[s], semw[s]).wait()
                pltpu.sync_copy(bufw.at[s], shw.at[idxb.at[j]], add=True)

        load(0, 0)

        @pl.loop(0, ROWS_PER_W // 2)
        def _(p):
            load(1, 2 * p + 1)
            scat(0, 2 * p)
            load(0, 2 * p + 2)
            scat(1, 2 * p + 1)

        plsc.subcore_barrier()
        pltpu.sync_copy(shw.at[pl.ds(sid * NSUB, NSUB)],
                        acc_hbm.at[cid, pl.ds(sid * NSUB, NSUB)])

    return sk(s2, dst_m, zeros)


def _prep_kv_mlp(p):
    w1 = p["W1"]
    return {
        "ea": w1[0:EDGE_DIM],
        "rf": w1[EDGE_DIM:EDGE_DIM + NG * EDGE_DIM],
        "hd": w1[84:84 + HID],
        "hs": w1[84 + HID:84 + 2 * HID],
        "iv": w1[84 + 2 * HID:],
        "b1": p["b1"].reshape(1, -1),
        "g": p["g"].reshape(1, -1),
        "be": p["be"].reshape(1, -1),
        "W2": p["W2"],
        "b2": p["b2"].reshape(1, -1),
    }


def _prep_q_mlp(p):
    return (p["W1"], p["b1"].reshape(1, -1), p["g"].reshape(1, -1),
            p["be"].reshape(1, -1), p["W2"], p["b2"].reshape(1, -1))


def kernel(h, x, edge_attr, edge_index, invar_ligand_shape, ligand_shape_emb,
           topo_out, e_w, params):
    del topo_out
    src = edge_index[0]
    dst = edge_index[1]
    dst_m = jnp.pad(dst.reshape(NROWS, CHUNK), ((0, NWORK * ROWS_PER_W - NROWS), (0, 0)))
    src_m = jnp.pad(src.reshape(NROWS, CHUNK), ((0, NWORK * ROWS_PER_W - NROWS), (0, 0)))
    dst_m64 = jnp.pad(dst.reshape(NCHTOT, CHUNK2), ((0, NWORK * NCH - NCHTOT), (0, 0)))
    src_m64 = jnp.pad(src.reshape(NCHTOT, CHUNK2), ((0, NWORK * NCH - NCHTOT), (0, 0)))
    ew = e_w.reshape(E, 1)
    xpad = jnp.pad(x, ((0, 0), (0, 13)))

    perm = np.array([(j % HEADS) * DH + j // HEADS for j in range(HID)],
                    dtype=np.int32)

    px = params["x2h"]
    hk = _prep_kv_mlp(px["hk"])
    hv = _prep_kv_mlp(px["hv"])
    hq = _prep_q_mlp(px["hq"])
    no = px["node_out"]
    n_w1 = jnp.concatenate([no["W1"][0:HID][perm], no["W1"][HID:]], axis=0)
    wn = (n_w1, no["b1"].reshape(1, -1), no["g"].reshape(1, -1),
          no["be"].reshape(1, -1), no["W2"], no["b2"].reshape(1, -1))

    ph = params["h2x"]
    xk = _prep_kv_mlp(ph["xk"])
    xv = _prep_kv_mlp(ph["xv"])
    xq = _prep_q_mlp(ph["xq"])
    wft = ph["Wf"].T
    wdt = ph["Wd"].T

    def kv_pack(m):
        return (m["hd"], m["iv"], m["b1"], m["hs"])

    xw = _gather_rel(xpad, dst_m, src_m)

    td1, ts1 = _node_tables(h, invar_ligand_shape,
                            kv_pack(hk), kv_pack(hv), hq)
    gd1, gs1 = _gather(td1, ts1, dst_m64, src_m64)
    hv_w2p = hv["W2"][:, perm]
    hv_b2p = hv["b2"][:, perm]
    s1w, s1d = _edge1(gd1, gs1, xw, edge_attr, ew,
                      (hk["ea"], hk["rf"], hk["g"], hk["be"],
                       hk["W2"], hk["b2"]),
                      (hv["ea"], hv["rf"], hv["g"], hv["be"],
                       hv_w2p, hv_b2p))
    accw1 = _scatter2(s1w, dst_m)
    accd1 = _scatter_den(s1d, dst_m)

    h_out, td2, ts2 = _node2(accw1, accd1, h, invar_ligand_shape, wn,
                             kv_pack(xk), kv_pack(xv), xq)

    gd2, gs2 = _gather(td2, ts2, dst_m64, src_m64)
    s2 = _edge2(gd2, gs2, xw, edge_attr, ew,
                (xk["ea"], xk["rf"], xk["g"], xk["be"], xk["W2"], xk["b2"]),
                (xv["ea"], xv["rf"], xv["g"], xv["be"], xv["W2"],
                 xv["b2"]))
    acc2 = _scatter2(s2, dst_m)

    se0 = ligand_shape_emb[:, :, 0]
    se1 = ligand_shape_emb[:, :, 1]
    se2 = ligand_shape_emb[:, :, 2]
    x_out = _tail(acc2, x, se0, se1, se2, wft, wdt)
    return h_out, x_out
